# baseline (speedup 1.0000x reference)
import functools
import math
from collections import defaultdict
from itertools import combinations

import numpy as np
import jax
import jax.numpy as jnp
from jax import lax
from jax.experimental import pallas as pl
from jax.experimental.pallas import tpu as pltpu
from jax.experimental.pallas import tpu_sc as plsc

_N_ATOMS = 100
_N_FEATURES = 64
_BATCH = 128
_BINS = 64
_STEP = 2.0 / (_BINS - 1)
_ISTEP = 1.0 / _STEP
_INV2PI = 1.0 / (2.0 * math.pi)

_NW = 32
_FRAMES_PER_W = _BATCH // _NW
_GROWS = 120
_GCOLS = 66
_WIN = 13


def _schedule_segments():
    pairs = np.array(list(combinations(range(_N_ATOMS - 1), 2)), dtype=np.int64)
    pairs = pairs[pairs[:, 1] - pairs[:, 0] != 1]
    buckets = defaultdict(list)
    for idx, (_, j) in enumerate(pairs):
        buckets[j].append(idx)
    chunks = []
    while buckets:
        js = sorted(buckets, key=lambda j: -len(buckets[j]))[:16]
        chunks.append([buckets[j].pop() for j in js])
        for j in js:
            if not buckets[j]:
                del buckets[j]
    si, sj = [], []
    for chunk in chunks:
        ci = [int(pairs[s, 0]) for s in chunk]
        cj = [int(pairs[s, 1]) for s in chunk]
        for lane in range(len(chunk), 16):
            ci.append(0)
            cj.append(100 + lane)
        si.extend(ci)
        sj.extend(cj)
    return (np.asarray(si, np.int32), np.asarray(sj, np.int32), len(chunks))


_SEG_I, _SEG_J, _NCHUNK = _schedule_segments()


def _rsqrt16(v):
    y = plsc.bitcast(jnp.int32(0x5F3759DF) - (plsc.bitcast(v, jnp.int32) >> 1),
                     jnp.float32)
    for _ in range(3):
        y = y * (1.5 - 0.5 * v * y * y)
    return y


def _asin16(x):
    ax = jnp.abs(x)
    p = jnp.full((16,), -0.0012624911, jnp.float32)
    for c in (0.0066700901, -0.0170881256, 0.0308918810,
              -0.0501743046, 0.0889789874, -0.2145988016, 1.5707963050):
        p = p * ax + jnp.float32(c)
    t = jnp.maximum(1.0 - ax, 0.0)
    s = t * _rsqrt16(t)
    return jnp.sign(x) * (jnp.float32(1.5707963267948966) - s * p)


def _cross(a, b):
    return (a[1] * b[2] - a[2] * b[1],
            a[2] * b[0] - a[0] * b[2],
            a[0] * b[1] - a[1] * b[0])


def _dot(a, b):
    return a[0] * b[0] + a[1] * b[1] + a[2] * b[2]


def _normed(v):
    r = _rsqrt16(_dot(v, v))
    return (v[0] * r, v[1] * r, v[2] * r)


def _sc_stage(segi, segj, xx, xy, xz, zg):
    mesh = plsc.VectorSubcoreMesh(core_axis_name="c", subcore_axis_name="s")

    @functools.partial(
        pl.kernel, mesh=mesh,
        compiler_params=pltpu.CompilerParams(needs_layout_passes=False),
        out_type=jax.ShapeDtypeStruct((_BATCH, _GROWS, _GCOLS), jnp.float32),
        scratch_types=[
            pltpu.VMEM((_NCHUNK * 16,), jnp.int32),
            pltpu.VMEM((_NCHUNK * 16,), jnp.int32),
            pltpu.VMEM((128,), jnp.float32),
            pltpu.VMEM((128,), jnp.float32),
            pltpu.VMEM((128,), jnp.float32),
            pltpu.VMEM((_GROWS, _GCOLS), jnp.float32),
        ],
    )
    def sc_g(segi_hbm, segj_hbm, xx_hbm, xy_hbm, xz_hbm, zg_hbm, g_hbm,
             segi_v, segj_v, xxv, xyv, xzv, gv):
        wid = lax.axis_index("s") * 2 + lax.axis_index("c")
        pltpu.sync_copy(segi_hbm, segi_v)
        pltpu.sync_copy(segj_hbm, segj_v)

        def frame_body(fi, carry):
            f = wid * _FRAMES_PER_W + fi
            pltpu.sync_copy(xx_hbm.at[f], xxv)
            pltpu.sync_copy(xy_hbm.at[f], xyv)
            pltpu.sync_copy(xz_hbm.at[f], xzv)
            pltpu.sync_copy(zg_hbm, gv)

            def chunk_body(c, carry2):
                base = c * 16
                ii = segi_v[pl.ds(base, 16)]
                jj = segj_v[pl.ds(base, 16)]
                i1 = ii + 1
                j1 = jj + 1
                pi = tuple(plsc.load_gather(r, [ii]) for r in (xxv, xyv, xzv))
                pi1 = tuple(plsc.load_gather(r, [i1]) for r in (xxv, xyv, xzv))
                pj = tuple(plsc.load_gather(r, [jj]) for r in (xxv, xyv, xzv))
                pj1 = tuple(plsc.load_gather(r, [j1]) for r in (xxv, xyv, xzv))

                dx0 = tuple(pj[a] - pi[a] for a in range(3))
                dx1 = tuple(pj1[a] - pi[a] for a in range(3))
                dx2 = tuple(pj[a] - pi1[a] for a in range(3))
                dx3 = tuple(pj1[a] - pi1[a] for a in range(3))
                r2a = _dot(dx0, dx0)
                r2b = _dot(dx3, dx3)
                w1 = jnp.exp(-r2a)
                w2 = jnp.exp(-r2b)

                ra = _rsqrt16(r2a)
                rb = _rsqrt16(r2b)
                u0 = tuple(dx0[a] * ra for a in range(3))
                u3 = tuple(dx3[a] * rb for a in range(3))
                u1 = _normed(dx1)
                u2 = _normed(dx2)

                c0 = _normed(_cross(u0, u1))
                c1 = _normed(_cross(u1, u3))
                c2 = _normed(_cross(u3, u2))
                c3 = _normed(_cross(u2, u0))
                omega = (_asin16(jnp.clip(_dot(c0, c1), -1.0, 1.0))
                         + _asin16(jnp.clip(_dot(c1, c2), -1.0, 1.0))
                         + _asin16(jnp.clip(_dot(c2, c3), -1.0, 1.0))
                         + _asin16(jnp.clip(_dot(c3, c0), -1.0, 1.0)))
                tj = tuple(pj1[a] - pj[a] for a in range(3))
                ti = tuple(pi1[a] - pi[a] for a in range(3))
                sgn = jnp.sign(_dot(_cross(tj, ti), u0))
                vb = omega * sgn * jnp.float32(_INV2PI * _ISTEP) + jnp.float32(31.5)
                k0 = lax.convert_element_type(vb + 0.5, jnp.int32)

                col64 = jnp.full((16,), 64, jnp.int32)
                plsc.addupdate_scatter(gv, [jj, col64], w1)
                plsc.addupdate_scatter(gv, [j1, col64], w2)
                for kk in range(_WIN):
                    k = k0 + (kk - _WIN // 2)
                    valid = (k >= 0) & (k <= 63)
                    d = vb - lax.convert_element_type(k, jnp.float32)
                    e = jnp.exp(-(d * d))
                    plsc.addupdate_scatter(gv, [jj, k], w1 * e, mask=valid)
                    plsc.addupdate_scatter(gv, [j1, k], w2 * e, mask=valid)
                return carry2

            lax.fori_loop(0, _NCHUNK, chunk_body, 0)
            pltpu.sync_copy(gv, g_hbm.at[f])
            return carry

        lax.fori_loop(0, _FRAMES_PER_W, frame_body, 0)

    return sc_g(segi, segj, xx, xy, xz, zg)


def _tc_finish_body(g_ref, nodef_ref, basis_ref, out_ref):
    g = g_ref[0]
    den = g[:, 64:65]
    dinv = jnp.where(den > 0, jnp.float32(1.0 / 1.12) / den, 0.0)
    gs = g[:, :_BINS] * dinv
    msg = lax.dot_general(gs, basis_ref[...], (((1,), (0,)), ((), ())),
                          preferred_element_type=jnp.float32)
    out_ref[0] = nodef_ref[0] + msg[:_N_ATOMS, :]


@jax.jit
def kernel(x, invariant_node_features, basis):
    xr = jnp.pad(x.reshape(_BATCH, _N_ATOMS, 3), ((0, 0), (0, 28), (0, 0)))
    xx, xy, xz = xr[:, :, 0], xr[:, :, 1], xr[:, :, 2]
    segi = jnp.asarray(_SEG_I)
    segj = jnp.asarray(_SEG_J)
    zg = jnp.zeros((_GROWS, _GCOLS), jnp.float32)

    g = _sc_stage(segi, segj, xx, xy, xz, zg)

    nodef3 = invariant_node_features.reshape(_BATCH, _N_ATOMS, _N_FEATURES)
    basis2 = basis[0, 0]
    out3 = pl.pallas_call(
        _tc_finish_body,
        grid=(_BATCH,),
        in_specs=[
            pl.BlockSpec((1, _GROWS, _GCOLS), lambda b: (b, 0, 0)),
            pl.BlockSpec((1, _N_ATOMS, _N_FEATURES), lambda b: (b, 0, 0)),
            pl.BlockSpec((_BINS, _N_FEATURES), lambda b: (0, 0)),
        ],
        out_specs=pl.BlockSpec((1, _N_ATOMS, _N_FEATURES), lambda b: (b, 0, 0)),
        out_shape=jax.ShapeDtypeStruct((_BATCH, _N_ATOMS, _N_FEATURES), jnp.float32),
    )(g, nodef3, basis2)
    return out3.reshape(_BATCH * _N_ATOMS, _N_FEATURES)


# speedup vs baseline: 1.5680x; 1.5680x over previous
import functools
import math
from collections import defaultdict
from itertools import combinations

import numpy as np
import jax
import jax.numpy as jnp
from jax import lax
from jax.experimental import pallas as pl
from jax.experimental.pallas import tpu as pltpu
from jax.experimental.pallas import tpu_sc as plsc

_N_ATOMS = 100
_N_FEATURES = 64
_BATCH = 128
_BINS = 64
_STEP = 2.0 / (_BINS - 1)
_ISTEP = 1.0 / _STEP
_CKS = [(-1.0 + k * _STEP) * _ISTEP for k in range(_BINS)]
_INV2PI = 1.0 / (2.0 * math.pi)

_NW = 32
_F_SC = 64
_F_TC = _BATCH - _F_SC
_FRAMES_PER_W = _F_SC // _NW
_GROWS = 120
_GCOLS = 66
_WIN = 13


def _schedule_segments():
    pairs = np.array(list(combinations(range(_N_ATOMS - 1), 2)), dtype=np.int64)
    pairs = pairs[pairs[:, 1] - pairs[:, 0] != 1]
    buckets = defaultdict(list)
    for idx, (_, j) in enumerate(pairs):
        buckets[j].append(idx)
    chunks = []
    while buckets:
        js = sorted(buckets, key=lambda j: -len(buckets[j]))[:16]
        chunks.append([buckets[j].pop() for j in js])
        for j in js:
            if not buckets[j]:
                del buckets[j]
    si, sj = [], []
    for chunk in chunks:
        ci = [int(pairs[s, 0]) for s in chunk]
        cj = [int(pairs[s, 1]) for s in chunk]
        for lane in range(len(chunk), 16):
            ci.append(0)
            cj.append(100 + lane)
        si.extend(ci)
        sj.extend(cj)
    return (np.asarray(si, np.int32), np.asarray(sj, np.int32), len(chunks))


_SEG_I, _SEG_J, _NCHUNK = _schedule_segments()


def _rsqrt16(v):
    y = plsc.bitcast(jnp.int32(0x5F3759DF) - (plsc.bitcast(v, jnp.int32) >> 1),
                     jnp.float32)
    for _ in range(3):
        y = y * (1.5 - 0.5 * v * y * y)
    return y


def _asin16(x):
    ax = jnp.abs(x)
    p = jnp.full((16,), -0.0012624911, jnp.float32)
    for c in (0.0066700901, -0.0170881256, 0.0308918810,
              -0.0501743046, 0.0889789874, -0.2145988016, 1.5707963050):
        p = p * ax + jnp.float32(c)
    t = jnp.maximum(1.0 - ax, 0.0)
    s = t * _rsqrt16(t)
    return jnp.sign(x) * (jnp.float32(1.5707963267948966) - s * p)


def _cross(a, b):
    return (a[1] * b[2] - a[2] * b[1],
            a[2] * b[0] - a[0] * b[2],
            a[0] * b[1] - a[1] * b[0])


def _dot(a, b):
    return a[0] * b[0] + a[1] * b[1] + a[2] * b[2]


def _normed(v):
    r = _rsqrt16(_dot(v, v))
    return (v[0] * r, v[1] * r, v[2] * r)


def _sc_stage(segi, segj, xx, xy, xz, zg):
    mesh = plsc.VectorSubcoreMesh(core_axis_name="c", subcore_axis_name="s")

    @functools.partial(
        pl.kernel, mesh=mesh,
        compiler_params=pltpu.CompilerParams(needs_layout_passes=False),
        out_type=jax.ShapeDtypeStruct((_F_SC, _GROWS, _GCOLS), jnp.float32),
        scratch_types=[
            pltpu.VMEM((_NCHUNK * 16,), jnp.int32),
            pltpu.VMEM((_NCHUNK * 16,), jnp.int32),
            pltpu.VMEM((128,), jnp.float32),
            pltpu.VMEM((128,), jnp.float32),
            pltpu.VMEM((128,), jnp.float32),
            pltpu.VMEM((_GROWS, _GCOLS), jnp.float32),
        ],
    )
    def sc_g(segi_hbm, segj_hbm, xx_hbm, xy_hbm, xz_hbm, zg_hbm, g_hbm,
             segi_v, segj_v, xxv, xyv, xzv, gv):
        wid = lax.axis_index("s") * 2 + lax.axis_index("c")
        pltpu.sync_copy(segi_hbm, segi_v)
        pltpu.sync_copy(segj_hbm, segj_v)

        def frame_body(fi, carry):
            f = wid * _FRAMES_PER_W + fi
            pltpu.sync_copy(xx_hbm.at[f], xxv)
            pltpu.sync_copy(xy_hbm.at[f], xyv)
            pltpu.sync_copy(xz_hbm.at[f], xzv)
            pltpu.sync_copy(zg_hbm, gv)

            def chunk_body(c, carry2):
                base = c * 16
                ii = segi_v[pl.ds(base, 16)]
                jj = segj_v[pl.ds(base, 16)]
                i1 = ii + 1
                j1 = jj + 1
                pi = tuple(plsc.load_gather(r, [ii]) for r in (xxv, xyv, xzv))
                pi1 = tuple(plsc.load_gather(r, [i1]) for r in (xxv, xyv, xzv))
                pj = tuple(plsc.load_gather(r, [jj]) for r in (xxv, xyv, xzv))
                pj1 = tuple(plsc.load_gather(r, [j1]) for r in (xxv, xyv, xzv))

                dx0 = tuple(pj[a] - pi[a] for a in range(3))
                dx1 = tuple(pj1[a] - pi[a] for a in range(3))
                dx2 = tuple(pj[a] - pi1[a] for a in range(3))
                dx3 = tuple(pj1[a] - pi1[a] for a in range(3))
                r2a = _dot(dx0, dx0)
                r2b = _dot(dx3, dx3)
                w1 = jnp.exp(-r2a)
                w2 = jnp.exp(-r2b)

                ra = _rsqrt16(r2a)
                rb = _rsqrt16(r2b)
                u0 = tuple(dx0[a] * ra for a in range(3))
                u3 = tuple(dx3[a] * rb for a in range(3))
                u1 = _normed(dx1)
                u2 = _normed(dx2)

                c0 = _normed(_cross(u0, u1))
                c1 = _normed(_cross(u1, u3))
                c2 = _normed(_cross(u3, u2))
                c3 = _normed(_cross(u2, u0))
                omega = (_asin16(jnp.clip(_dot(c0, c1), -1.0, 1.0))
                         + _asin16(jnp.clip(_dot(c1, c2), -1.0, 1.0))
                         + _asin16(jnp.clip(_dot(c2, c3), -1.0, 1.0))
                         + _asin16(jnp.clip(_dot(c3, c0), -1.0, 1.0)))
                tj = tuple(pj1[a] - pj[a] for a in range(3))
                ti = tuple(pi1[a] - pi[a] for a in range(3))
                sgn = jnp.sign(_dot(_cross(tj, ti), u0))
                vb = omega * sgn * jnp.float32(_INV2PI * _ISTEP) + jnp.float32(31.5)
                k0 = lax.convert_element_type(vb + 0.5, jnp.int32)

                col64 = jnp.full((16,), 64, jnp.int32)
                plsc.addupdate_scatter(gv, [jj, col64], w1)
                plsc.addupdate_scatter(gv, [j1, col64], w2)
                for kk in range(_WIN):
                    k = k0 + (kk - _WIN // 2)
                    valid = (k >= 0) & (k <= 63)
                    d = vb - lax.convert_element_type(k, jnp.float32)
                    e = jnp.exp(-(d * d))
                    plsc.addupdate_scatter(gv, [jj, k], w1 * e, mask=valid)
                    plsc.addupdate_scatter(gv, [j1, k], w2 * e, mask=valid)
                return carry2

            lax.fori_loop(0, _NCHUNK, chunk_body, 0)
            pltpu.sync_copy(gv, g_hbm.at[f])
            return carry

        lax.fori_loop(0, _FRAMES_PER_W, frame_body, 0)

    return sc_g(segi, segj, xx, xy, xz, zg)


def _shl_lane(m):
    return jnp.concatenate([m[:, 1:], m[:, :1]], axis=1)


def _shl_sub(m):
    return jnp.concatenate([m[1:, :], m[:1, :]], axis=0)


def _shr_lane_row(v):
    return jnp.concatenate([jnp.zeros((1, 1), jnp.float32), v[:, :-1]], axis=1)


def _asin_tc(x):
    ax = jnp.abs(x)
    p = jnp.float32(-0.0012624911)
    for c in (0.0066700901, -0.0170881256, 0.0308918810,
              -0.0501743046, 0.0889789874, -0.2145988016, 1.5707963050):
        p = p * ax + jnp.float32(c)
    r = jnp.float32(1.5707963267948966) - jnp.sqrt(jnp.maximum(1.0 - ax, 0.0)) * p
    return jnp.sign(x) * r


def _cross_tc(a, b):
    return (a[1] * b[2] - a[2] * b[1],
            a[2] * b[0] - a[0] * b[2],
            a[0] * b[1] - a[1] * b[0])


def _norm3_tc(v):
    r = lax.rsqrt(v[0] * v[0] + v[1] * v[1] + v[2] * v[2])
    return (v[0] * r, v[1] * r, v[2] * r)


def _dot3_tc(a, b):
    return a[0] * b[0] + a[1] * b[1] + a[2] * b[2]


def _dense_body(xt_ref, nodef_ref, basis_ref, out_ref, gt_ref):
    xr = xt_ref[0]
    rows = [jnp.broadcast_to(xr[c:c + 1, :], (128, 128)) for c in range(3)]
    cols = [jnp.transpose(r) for r in rows]
    d = [rows[c] - cols[c] for c in range(3)]
    r2 = d[0] * d[0] + d[1] * d[1] + d[2] * d[2]
    w = jnp.exp(-r2)
    inv = lax.rsqrt(r2)
    ua = tuple(d[c] * inv for c in range(3))
    ub = tuple(_shl_lane(u) for u in ua)
    uc = tuple(_shl_sub(u) for u in ua)
    ud = tuple(_shl_sub(u) for u in ub)

    c0 = _norm3_tc(_cross_tc(ua, ub))
    c1 = _norm3_tc(_cross_tc(ub, ud))
    c2 = _norm3_tc(_cross_tc(ud, uc))
    c3 = _norm3_tc(_cross_tc(uc, ua))
    omega = (_asin_tc(jnp.clip(_dot3_tc(c0, c1), -1.0, 1.0))
             + _asin_tc(jnp.clip(_dot3_tc(c1, c2), -1.0, 1.0))
             + _asin_tc(jnp.clip(_dot3_tc(c2, c3), -1.0, 1.0))
             + _asin_tc(jnp.clip(_dot3_tc(c3, c0), -1.0, 1.0)))

    tj = tuple(_shl_lane(rows[c]) - rows[c] for c in range(3))
    ti = tuple(_shl_sub(cols[c]) - cols[c] for c in range(3))
    sgn = jnp.sign(_dot3_tc(_cross_tc(tj, ti), ua))
    wr = omega * sgn * jnp.float32(_INV2PI)

    ii = lax.broadcasted_iota(jnp.int32, (128, 128), 0)
    jj = lax.broadcasted_iota(jnp.int32, (128, 128), 1)
    mask = (ii + 2 <= jj) & (jj <= 98)
    wrs = jnp.where(mask, wr * jnp.float32(_ISTEP), 0.0)
    w1 = jnp.where(mask, w, 0.0)
    w2 = jnp.where(mask, _shl_sub(_shl_lane(w)), 0.0)

    s1 = jnp.sum(w1, axis=0, keepdims=True)
    s2 = jnp.sum(w2, axis=0, keepdims=True)
    denom = s1 + _shr_lane_row(s2)
    dinv = jnp.where(denom > 0, jnp.float32(1.0 / 1.12) / denom, 0.0)

    for k in range(_BINS):
        dk = wrs - jnp.float32(_CKS[k])
        e = jnp.exp(-(dk * dk))
        r1 = jnp.sum(w1 * e, axis=0, keepdims=True)
        r2v = jnp.sum(w2 * e, axis=0, keepdims=True)
        gt_ref[k:k + 1, :] = r1 + _shr_lane_row(r2v)

    gt = gt_ref[...] * dinv
    msg = lax.dot_general(gt, basis_ref[...], (((0,), (0,)), ((), ())),
                          preferred_element_type=jnp.float32)
    out_ref[0] = nodef_ref[0] + msg[:_N_ATOMS, :]


def _tc_finish_body(g_ref, nodef_ref, basis_ref, out_ref):
    g = g_ref[0]
    den = g[:, 64:65]
    dinv = jnp.where(den > 0, jnp.float32(1.0 / 1.12) / den, 0.0)
    gs = g[:, :_BINS] * dinv
    msg = lax.dot_general(gs, basis_ref[...], (((1,), (0,)), ((), ())),
                          preferred_element_type=jnp.float32)
    out_ref[0] = nodef_ref[0] + msg[:_N_ATOMS, :]


@jax.jit
def kernel(x, invariant_node_features, basis):
    xr3 = x.reshape(_BATCH, _N_ATOMS, 3)
    nodef3 = invariant_node_features.reshape(_BATCH, _N_ATOMS, _N_FEATURES)
    basis2 = basis[0, 0]

    xpad = jnp.pad(xr3[:_F_SC], ((0, 0), (0, 28), (0, 0)))
    xx, xy, xz = xpad[:, :, 0], xpad[:, :, 1], xpad[:, :, 2]
    zg = jnp.zeros((_GROWS, _GCOLS), jnp.float32)
    g = _sc_stage(jnp.asarray(_SEG_I), jnp.asarray(_SEG_J), xx, xy, xz, zg)

    xt = jnp.transpose(xr3[_F_SC:], (0, 2, 1))
    xt = jnp.pad(xt, ((0, 0), (0, 0), (0, 128 - _N_ATOMS)))
    out_tc = pl.pallas_call(
        _dense_body,
        grid=(_F_TC,),
        in_specs=[
            pl.BlockSpec((1, 3, 128), lambda b: (b, 0, 0)),
            pl.BlockSpec((1, _N_ATOMS, _N_FEATURES), lambda b: (b + _F_SC, 0, 0)),
            pl.BlockSpec((_BINS, _N_FEATURES), lambda b: (0, 0)),
        ],
        out_specs=pl.BlockSpec((1, _N_ATOMS, _N_FEATURES), lambda b: (b, 0, 0)),
        out_shape=jax.ShapeDtypeStruct((_F_TC, _N_ATOMS, _N_FEATURES), jnp.float32),
        scratch_shapes=[pltpu.VMEM((_BINS, 128), jnp.float32)],
    )(xt, nodef3, basis2)

    out_sc = pl.pallas_call(
        _tc_finish_body,
        grid=(_F_SC,),
        in_specs=[
            pl.BlockSpec((1, _GROWS, _GCOLS), lambda b: (b, 0, 0)),
            pl.BlockSpec((1, _N_ATOMS, _N_FEATURES), lambda b: (b, 0, 0)),
            pl.BlockSpec((_BINS, _N_FEATURES), lambda b: (0, 0)),
        ],
        out_specs=pl.BlockSpec((1, _N_ATOMS, _N_FEATURES), lambda b: (b, 0, 0)),
        out_shape=jax.ShapeDtypeStruct((_F_SC, _N_ATOMS, _N_FEATURES), jnp.float32),
    )(g, nodef3, basis2)

    out3 = jnp.concatenate([out_sc, out_tc], axis=0)
    return out3.reshape(_BATCH * _N_ATOMS, _N_FEATURES)


# speedup vs baseline: 1.5973x; 1.0187x over previous
import functools
import math
from collections import defaultdict
from itertools import combinations

import numpy as np
import jax
import jax.numpy as jnp
from jax import lax
from jax.experimental import pallas as pl
from jax.experimental.pallas import tpu as pltpu
from jax.experimental.pallas import tpu_sc as plsc

_N_ATOMS = 100
_N_FEATURES = 64
_BATCH = 128
_BINS = 64
_STEP = 2.0 / (_BINS - 1)
_ISTEP = 1.0 / _STEP
_CKS = [(-1.0 + k * _STEP) * _ISTEP for k in range(_BINS)]
_INV2PI = 1.0 / (2.0 * math.pi)

_NW = 32
_F_SC = 80
_F_TC = _BATCH - _F_SC
_GROWS = 120
_GCOLS = 66
_WIN = 9
_FB = 8


def _schedule_segments():
    pairs = np.array(list(combinations(range(_N_ATOMS - 1), 2)), dtype=np.int64)
    pairs = pairs[pairs[:, 1] - pairs[:, 0] != 1]
    buckets = defaultdict(list)
    for idx, (_, j) in enumerate(pairs):
        buckets[j].append(idx)
    chunks = []
    while buckets:
        js = sorted(buckets, key=lambda j: -len(buckets[j]))[:16]
        chunks.append([buckets[j].pop() for j in js])
        for j in js:
            if not buckets[j]:
                del buckets[j]
    si, sj = [], []
    for chunk in chunks:
        ci = [int(pairs[s, 0]) for s in chunk]
        cj = [int(pairs[s, 1]) for s in chunk]
        for lane in range(len(chunk), 16):
            ci.append(0)
            cj.append(100 + lane)
        si.extend(ci)
        sj.extend(cj)
    return (np.asarray(si, np.int32), np.asarray(sj, np.int32), len(chunks))


_SEG_I, _SEG_J, _NCHUNK = _schedule_segments()


def _rsqrt16(v):
    y = plsc.bitcast(jnp.int32(0x5F3759DF) - (plsc.bitcast(v, jnp.int32) >> 1),
                     jnp.float32)
    for _ in range(3):
        y = y * (1.5 - 0.5 * v * y * y)
    return y


def _asin16(x):
    ax = jnp.abs(x)
    p = jnp.full((16,), -0.0012624911, jnp.float32)
    for c in (0.0066700901, -0.0170881256, 0.0308918810,
              -0.0501743046, 0.0889789874, -0.2145988016, 1.5707963050):
        p = p * ax + jnp.float32(c)
    t = jnp.maximum(1.0 - ax, 0.0)
    s = t * _rsqrt16(t)
    return jnp.sign(x) * (jnp.float32(1.5707963267948966) - s * p)


def _cross(a, b):
    return (a[1] * b[2] - a[2] * b[1],
            a[2] * b[0] - a[0] * b[2],
            a[0] * b[1] - a[1] * b[0])


def _dot(a, b):
    return a[0] * b[0] + a[1] * b[1] + a[2] * b[2]


def _normed(v):
    r = _rsqrt16(_dot(v, v))
    return (v[0] * r, v[1] * r, v[2] * r)


def _sc_stage(segi, segj, xx, xy, xz, zg):
    mesh = plsc.VectorSubcoreMesh(core_axis_name="c", subcore_axis_name="s")

    @functools.partial(
        pl.kernel, mesh=mesh,
        compiler_params=pltpu.CompilerParams(needs_layout_passes=False),
        out_type=jax.ShapeDtypeStruct((_F_SC, _GROWS, _GCOLS), jnp.float32),
        scratch_types=[
            pltpu.VMEM((_NCHUNK * 16,), jnp.int32),
            pltpu.VMEM((_NCHUNK * 16,), jnp.int32),
            pltpu.VMEM((128,), jnp.float32),
            pltpu.VMEM((128,), jnp.float32),
            pltpu.VMEM((128,), jnp.float32),
            pltpu.VMEM((_GROWS, _GCOLS), jnp.float32),
        ],
    )
    def sc_g(segi_hbm, segj_hbm, xx_hbm, xy_hbm, xz_hbm, zg_hbm, g_hbm,
             segi_v, segj_v, xxv, xyv, xzv, gv):
        wid = lax.axis_index("s") * 2 + lax.axis_index("c")
        pltpu.sync_copy(segi_hbm, segi_v)
        pltpu.sync_copy(segj_hbm, segj_v)
        fbase = wid * 2 + jnp.minimum(wid, 16)
        fcount = jnp.where(wid < 16, 3, 2)

        def frame_body(fi, carry):
            f = fbase + fi
            pltpu.sync_copy(xx_hbm.at[f], xxv)
            pltpu.sync_copy(xy_hbm.at[f], xyv)
            pltpu.sync_copy(xz_hbm.at[f], xzv)
            pltpu.sync_copy(zg_hbm, gv)

            def chunk_body(c, carry2):
                base = c * 16
                ii = segi_v[pl.ds(base, 16)]
                jj = segj_v[pl.ds(base, 16)]
                i1 = ii + 1
                j1 = jj + 1
                pi = tuple(plsc.load_gather(r, [ii]) for r in (xxv, xyv, xzv))
                pi1 = tuple(plsc.load_gather(r, [i1]) for r in (xxv, xyv, xzv))
                pj = tuple(plsc.load_gather(r, [jj]) for r in (xxv, xyv, xzv))
                pj1 = tuple(plsc.load_gather(r, [j1]) for r in (xxv, xyv, xzv))

                dx0 = tuple(pj[a] - pi[a] for a in range(3))
                dx1 = tuple(pj1[a] - pi[a] for a in range(3))
                dx2 = tuple(pj[a] - pi1[a] for a in range(3))
                dx3 = tuple(pj1[a] - pi1[a] for a in range(3))
                r2a = _dot(dx0, dx0)
                r2b = _dot(dx3, dx3)
                w1 = jnp.exp(-r2a)
                w2 = jnp.exp(-r2b)

                ra = _rsqrt16(r2a)
                rb = _rsqrt16(r2b)
                u0 = tuple(dx0[a] * ra for a in range(3))
                u3 = tuple(dx3[a] * rb for a in range(3))
                u1 = _normed(dx1)
                u2 = _normed(dx2)

                c0 = _normed(_cross(u0, u1))
                c1 = _normed(_cross(u1, u3))
                c2 = _normed(_cross(u3, u2))
                c3 = _normed(_cross(u2, u0))
                omega = (_asin16(jnp.clip(_dot(c0, c1), -1.0, 1.0))
                         + _asin16(jnp.clip(_dot(c1, c2), -1.0, 1.0))
                         + _asin16(jnp.clip(_dot(c2, c3), -1.0, 1.0))
                         + _asin16(jnp.clip(_dot(c3, c0), -1.0, 1.0)))
                tj = tuple(pj1[a] - pj[a] for a in range(3))
                ti = tuple(pi1[a] - pi[a] for a in range(3))
                sgn = jnp.sign(_dot(_cross(tj, ti), u0))
                vb = omega * sgn * jnp.float32(_INV2PI * _ISTEP) + jnp.float32(31.5)
                k0 = lax.convert_element_type(vb + 0.5, jnp.int32)

                col64 = jnp.full((16,), 64, jnp.int32)
                plsc.addupdate_scatter(gv, [jj, col64], w1)
                plsc.addupdate_scatter(gv, [j1, col64], w2)
                for kk in range(_WIN):
                    k = k0 + (kk - _WIN // 2)
                    valid = (k >= 0) & (k <= 63)
                    d = vb - lax.convert_element_type(k, jnp.float32)
                    e = jnp.exp(-(d * d))
                    plsc.addupdate_scatter(gv, [jj, k], w1 * e, mask=valid)
                    plsc.addupdate_scatter(gv, [j1, k], w2 * e, mask=valid)
                return carry2

            lax.fori_loop(0, _NCHUNK, chunk_body, 0)
            pltpu.sync_copy(gv, g_hbm.at[f])
            return carry

        lax.fori_loop(0, fcount, frame_body, 0)

    return sc_g(segi, segj, xx, xy, xz, zg)


def _shl_lane(m):
    return jnp.concatenate([m[:, 1:], m[:, :1]], axis=1)


def _shl_sub(m):
    return jnp.concatenate([m[1:, :], m[:1, :]], axis=0)


def _shr_lane_row(v):
    return jnp.concatenate([jnp.zeros((1, 1), jnp.float32), v[:, :-1]], axis=1)


def _asin_tc(x):
    ax = jnp.abs(x)
    p = jnp.float32(-0.0012624911)
    for c in (0.0066700901, -0.0170881256, 0.0308918810,
              -0.0501743046, 0.0889789874, -0.2145988016, 1.5707963050):
        p = p * ax + jnp.float32(c)
    r = jnp.float32(1.5707963267948966) - jnp.sqrt(jnp.maximum(1.0 - ax, 0.0)) * p
    return jnp.sign(x) * r


def _cross_tc(a, b):
    return (a[1] * b[2] - a[2] * b[1],
            a[2] * b[0] - a[0] * b[2],
            a[0] * b[1] - a[1] * b[0])


def _norm3_tc(v):
    r = lax.rsqrt(v[0] * v[0] + v[1] * v[1] + v[2] * v[2])
    return (v[0] * r, v[1] * r, v[2] * r)


def _dot3_tc(a, b):
    return a[0] * b[0] + a[1] * b[1] + a[2] * b[2]


def _dense_body(xt_ref, basis_ref, out_ref, gt_ref):
    xr = xt_ref[0]
    rows = [jnp.broadcast_to(xr[c:c + 1, :], (128, 128)) for c in range(3)]
    cols = [jnp.transpose(r) for r in rows]
    d = [rows[c] - cols[c] for c in range(3)]
    r2 = d[0] * d[0] + d[1] * d[1] + d[2] * d[2]
    w = jnp.exp(-r2)
    inv = lax.rsqrt(r2)
    ua = tuple(d[c] * inv for c in range(3))
    ub = tuple(_shl_lane(u) for u in ua)
    uc = tuple(_shl_sub(u) for u in ua)
    ud = tuple(_shl_sub(u) for u in ub)

    c0 = _norm3_tc(_cross_tc(ua, ub))
    c1 = _norm3_tc(_cross_tc(ub, ud))
    c2 = _norm3_tc(_cross_tc(ud, uc))
    c3 = _norm3_tc(_cross_tc(uc, ua))
    omega = (_asin_tc(jnp.clip(_dot3_tc(c0, c1), -1.0, 1.0))
             + _asin_tc(jnp.clip(_dot3_tc(c1, c2), -1.0, 1.0))
             + _asin_tc(jnp.clip(_dot3_tc(c2, c3), -1.0, 1.0))
             + _asin_tc(jnp.clip(_dot3_tc(c3, c0), -1.0, 1.0)))

    tj = tuple(_shl_lane(rows[c]) - rows[c] for c in range(3))
    ti = tuple(_shl_sub(cols[c]) - cols[c] for c in range(3))
    sgn = jnp.sign(_dot3_tc(_cross_tc(tj, ti), ua))
    wr = omega * sgn * jnp.float32(_INV2PI)

    ii = lax.broadcasted_iota(jnp.int32, (128, 128), 0)
    jj = lax.broadcasted_iota(jnp.int32, (128, 128), 1)
    mask = (ii + 2 <= jj) & (jj <= 98)
    wrs = jnp.where(mask, wr * jnp.float32(_ISTEP), 0.0)
    w1 = jnp.where(mask, w, 0.0)
    w2 = jnp.where(mask, _shl_sub(_shl_lane(w)), 0.0)

    s1 = jnp.sum(w1, axis=0, keepdims=True)
    s2 = jnp.sum(w2, axis=0, keepdims=True)
    denom = s1 + _shr_lane_row(s2)
    dinv = jnp.where(denom > 0, jnp.float32(1.0 / 1.12) / denom, 0.0)

    for k in range(_BINS):
        dk = wrs - jnp.float32(_CKS[k])
        e = jnp.exp(-(dk * dk))
        r1 = jnp.sum(w1 * e, axis=0, keepdims=True)
        r2v = jnp.sum(w2 * e, axis=0, keepdims=True)
        gt_ref[k:k + 1, :] = r1 + _shr_lane_row(r2v)

    gt = gt_ref[...] * dinv
    msg = lax.dot_general(gt, basis_ref[...], (((0,), (0,)), ((), ())),
                          preferred_element_type=jnp.float32)
    out_ref[0] = msg[:_N_ATOMS, :]


def _tc_finish_body(g_ref, basis_ref, out_ref):
    for i in range(_FB):
        g = g_ref[i]
        den = g[:, 64:65]
        dinv = jnp.where(den > 0, jnp.float32(1.0 / 1.12) / den, 0.0)
        gs = g[:, :_BINS] * dinv
        msg = lax.dot_general(gs, basis_ref[...], (((1,), (0,)), ((), ())),
                              preferred_element_type=jnp.float32)
        out_ref[i] = msg[:_N_ATOMS, :]


@jax.jit
def kernel(x, invariant_node_features, basis):
    xr3 = x.reshape(_BATCH, _N_ATOMS, 3)
    basis2 = basis[0, 0]

    xpad = jnp.pad(xr3[:_F_SC], ((0, 0), (0, 28), (0, 0)))
    xx, xy, xz = xpad[:, :, 0], xpad[:, :, 1], xpad[:, :, 2]
    zg = jnp.zeros((_GROWS, _GCOLS), jnp.float32)
    g = _sc_stage(jnp.asarray(_SEG_I), jnp.asarray(_SEG_J), xx, xy, xz, zg)

    xt = jnp.transpose(xr3[_F_SC:], (0, 2, 1))
    xt = jnp.pad(xt, ((0, 0), (0, 0), (0, 128 - _N_ATOMS)))
    msg_tc = pl.pallas_call(
        _dense_body,
        grid=(_F_TC,),
        in_specs=[
            pl.BlockSpec((1, 3, 128), lambda b: (b, 0, 0)),
            pl.BlockSpec((_BINS, _N_FEATURES), lambda b: (0, 0)),
        ],
        out_specs=pl.BlockSpec((1, _N_ATOMS, _N_FEATURES), lambda b: (b, 0, 0)),
        out_shape=jax.ShapeDtypeStruct((_F_TC, _N_ATOMS, _N_FEATURES), jnp.float32),
        scratch_shapes=[pltpu.VMEM((_BINS, 128), jnp.float32)],
    )(xt, basis2)

    msg_sc = pl.pallas_call(
        _tc_finish_body,
        grid=(_F_SC // _FB,),
        in_specs=[
            pl.BlockSpec((_FB, _GROWS, _GCOLS), lambda b: (b, 0, 0)),
            pl.BlockSpec((_BINS, _N_FEATURES), lambda b: (0, 0)),
        ],
        out_specs=pl.BlockSpec((_FB, _N_ATOMS, _N_FEATURES), lambda b: (b, 0, 0)),
        out_shape=jax.ShapeDtypeStruct((_F_SC, _N_ATOMS, _N_FEATURES), jnp.float32),
    )(g, basis2)

    msg = jnp.concatenate([msg_sc, msg_tc], axis=0).reshape(
        _BATCH * _N_ATOMS, _N_FEATURES)
    return invariant_node_features + msg


# speedup vs baseline: 1.8974x; 1.1878x over previous
import functools
import math
from collections import defaultdict
from itertools import combinations

import numpy as np
import jax
import jax.numpy as jnp
from jax import lax
from jax.experimental import pallas as pl
from jax.experimental.pallas import tpu as pltpu
from jax.experimental.pallas import tpu_sc as plsc

_N_ATOMS = 100
_N_FEATURES = 64
_BATCH = 128
_BINS = 64
_STEP = 2.0 / (_BINS - 1)
_ISTEP = 1.0 / _STEP
_CKS = [(-1.0 + k * _STEP) * _ISTEP for k in range(_BINS)]
_INV2PI = 1.0 / (2.0 * math.pi)

_NW = 32
_F_SC = 64
_F_TC = _BATCH - _F_SC
_GROWS = 120
_GCOLS = 66
_WIN = 9
_FB = 8


def _schedule_segments():
    pairs = np.array(list(combinations(range(_N_ATOMS - 1), 2)), dtype=np.int64)
    pairs = pairs[pairs[:, 1] - pairs[:, 0] != 1]
    buckets = defaultdict(list)
    for idx, (_, j) in enumerate(pairs):
        buckets[j].append(idx)
    chunks = []
    while buckets:
        js = sorted(buckets, key=lambda j: -len(buckets[j]))[:16]
        chunks.append([buckets[j].pop() for j in js])
        for j in js:
            if not buckets[j]:
                del buckets[j]
    si, sj = [], []
    for chunk in chunks:
        ci = [int(pairs[s, 0]) for s in chunk]
        cj = [int(pairs[s, 1]) for s in chunk]
        for lane in range(len(chunk), 16):
            ci.append(0)
            cj.append(100 + lane)
        si.extend(ci)
        sj.extend(cj)
    return (np.asarray(si, np.int32), np.asarray(sj, np.int32), len(chunks))


_SEG_I, _SEG_J, _NCHUNK = _schedule_segments()


def _rsqrt16(v):
    y = plsc.bitcast(jnp.int32(0x5F3759DF) - (plsc.bitcast(v, jnp.int32) >> 1),
                     jnp.float32)
    for _ in range(3):
        y = y * (1.5 - 0.5 * v * y * y)
    return y


def _asin16(x):
    ax = jnp.abs(x)
    p = jnp.full((16,), -0.0012624911, jnp.float32)
    for c in (0.0066700901, -0.0170881256, 0.0308918810,
              -0.0501743046, 0.0889789874, -0.2145988016, 1.5707963050):
        p = p * ax + jnp.float32(c)
    t = jnp.maximum(1.0 - ax, 0.0)
    s = t * _rsqrt16(t)
    return jnp.sign(x) * (jnp.float32(1.5707963267948966) - s * p)


def _cross(a, b):
    return (a[1] * b[2] - a[2] * b[1],
            a[2] * b[0] - a[0] * b[2],
            a[0] * b[1] - a[1] * b[0])


def _dot(a, b):
    return a[0] * b[0] + a[1] * b[1] + a[2] * b[2]


def _normed(v):
    r = _rsqrt16(_dot(v, v))
    return (v[0] * r, v[1] * r, v[2] * r)


def _sc_stage(segi, segj, xx, xy, xz, zg):
    mesh = plsc.VectorSubcoreMesh(core_axis_name="c", subcore_axis_name="s")

    @functools.partial(
        pl.kernel, mesh=mesh,
        compiler_params=pltpu.CompilerParams(needs_layout_passes=False),
        out_type=jax.ShapeDtypeStruct((_F_SC, _GROWS, _GCOLS), jnp.float32),
        scratch_types=[
            pltpu.VMEM((_NCHUNK * 16,), jnp.int32),
            pltpu.VMEM((_NCHUNK * 16,), jnp.int32),
            pltpu.VMEM((128,), jnp.float32),
            pltpu.VMEM((128,), jnp.float32),
            pltpu.VMEM((128,), jnp.float32),
            pltpu.VMEM((_GROWS, _GCOLS), jnp.float32),
        ],
    )
    def sc_g(segi_hbm, segj_hbm, xx_hbm, xy_hbm, xz_hbm, zg_hbm, g_hbm,
             segi_v, segj_v, xxv, xyv, xzv, gv):
        wid = lax.axis_index("s") * 2 + lax.axis_index("c")
        pltpu.sync_copy(segi_hbm, segi_v)
        pltpu.sync_copy(segj_hbm, segj_v)
        nf = _F_SC // _NW

        def frame_body(fi, carry):
            f = wid * nf + fi
            pltpu.sync_copy(xx_hbm.at[f], xxv)
            pltpu.sync_copy(xy_hbm.at[f], xyv)
            pltpu.sync_copy(xz_hbm.at[f], xzv)
            pltpu.sync_copy(zg_hbm, gv)

            def chunk_body(c, carry2):
                base = c * 16
                ii = segi_v[pl.ds(base, 16)]
                jj = segj_v[pl.ds(base, 16)]
                i1 = ii + 1
                j1 = jj + 1
                pi = tuple(plsc.load_gather(r, [ii]) for r in (xxv, xyv, xzv))
                pi1 = tuple(plsc.load_gather(r, [i1]) for r in (xxv, xyv, xzv))
                pj = tuple(plsc.load_gather(r, [jj]) for r in (xxv, xyv, xzv))
                pj1 = tuple(plsc.load_gather(r, [j1]) for r in (xxv, xyv, xzv))

                dx0 = tuple(pj[a] - pi[a] for a in range(3))
                dx1 = tuple(pj1[a] - pi[a] for a in range(3))
                dx2 = tuple(pj[a] - pi1[a] for a in range(3))
                dx3 = tuple(pj1[a] - pi1[a] for a in range(3))
                r2a = _dot(dx0, dx0)
                r2b = _dot(dx3, dx3)
                w1 = jnp.exp(-r2a)
                w2 = jnp.exp(-r2b)

                ra = _rsqrt16(r2a)
                rb = _rsqrt16(r2b)
                u0 = tuple(dx0[a] * ra for a in range(3))
                u3 = tuple(dx3[a] * rb for a in range(3))
                u1 = _normed(dx1)
                u2 = _normed(dx2)

                c0 = _normed(_cross(u0, u1))
                c1 = _normed(_cross(u1, u3))
                c2 = _normed(_cross(u3, u2))
                c3 = _normed(_cross(u2, u0))
                omega = (_asin16(jnp.clip(_dot(c0, c1), -1.0, 1.0))
                         + _asin16(jnp.clip(_dot(c1, c2), -1.0, 1.0))
                         + _asin16(jnp.clip(_dot(c2, c3), -1.0, 1.0))
                         + _asin16(jnp.clip(_dot(c3, c0), -1.0, 1.0)))
                tj = tuple(pj1[a] - pj[a] for a in range(3))
                ti = tuple(pi1[a] - pi[a] for a in range(3))
                sgn = jnp.sign(_dot(_cross(tj, ti), u0))
                vb = omega * sgn * jnp.float32(_INV2PI * _ISTEP) + jnp.float32(31.5)
                k0 = lax.convert_element_type(vb + 0.5, jnp.int32)

                col64 = jnp.full((16,), 64, jnp.int32)
                plsc.addupdate_scatter(gv, [jj, col64], w1)
                plsc.addupdate_scatter(gv, [j1, col64], w2)
                klo = k0 - _WIN // 2
                d0 = vb - lax.convert_element_type(klo, jnp.float32)
                e = jnp.exp(-(d0 * d0))
                ratio = jnp.exp(2.0 * d0 - 1.0)
                decay = jnp.full((16,), math.exp(-2.0), jnp.float32)
                for kk in range(_WIN):
                    k = klo + kk
                    valid = (k >= 0) & (k <= 63)
                    plsc.addupdate_scatter(gv, [jj, k], w1 * e, mask=valid)
                    plsc.addupdate_scatter(gv, [j1, k], w2 * e, mask=valid)
                    if kk < _WIN - 1:
                        e = e * ratio
                        ratio = ratio * decay
                return carry2

            lax.fori_loop(0, _NCHUNK, chunk_body, 0)
            pltpu.sync_copy(gv, g_hbm.at[f])
            return carry

        lax.fori_loop(0, nf, frame_body, 0)

    return sc_g(segi, segj, xx, xy, xz, zg)


def _shl_lane(m):
    return jnp.concatenate([m[:, 1:], m[:, :1]], axis=1)


def _shl_sub(m):
    return jnp.concatenate([m[1:, :], m[:1, :]], axis=0)


def _shr_lane_row(v):
    return jnp.concatenate([jnp.zeros((1, 1), jnp.float32), v[:, :-1]], axis=1)


def _asin_tc(x):
    ax = jnp.abs(x)
    p = jnp.float32(-0.0012624911)
    for c in (0.0066700901, -0.0170881256, 0.0308918810,
              -0.0501743046, 0.0889789874, -0.2145988016, 1.5707963050):
        p = p * ax + jnp.float32(c)
    r = jnp.float32(1.5707963267948966) - jnp.sqrt(jnp.maximum(1.0 - ax, 0.0)) * p
    return jnp.sign(x) * r


def _cross_tc(a, b):
    return (a[1] * b[2] - a[2] * b[1],
            a[2] * b[0] - a[0] * b[2],
            a[0] * b[1] - a[1] * b[0])


def _norm3_tc(v):
    r = lax.rsqrt(v[0] * v[0] + v[1] * v[1] + v[2] * v[2])
    return (v[0] * r, v[1] * r, v[2] * r)


def _dot3_tc(a, b):
    return a[0] * b[0] + a[1] * b[1] + a[2] * b[2]


def _dense_body(xt_ref, basis_ref, out_ref, gt_ref):
    xr = xt_ref[0]
    rows = [jnp.broadcast_to(xr[c:c + 1, :], (128, 128)) for c in range(3)]
    cols = [jnp.transpose(r) for r in rows]
    d = [rows[c] - cols[c] for c in range(3)]
    r2 = d[0] * d[0] + d[1] * d[1] + d[2] * d[2]
    w = jnp.exp(-r2)
    inv = lax.rsqrt(r2)
    ua = tuple(d[c] * inv for c in range(3))
    ub = tuple(_shl_lane(u) for u in ua)
    uc = tuple(_shl_sub(u) for u in ua)
    ud = tuple(_shl_sub(u) for u in ub)

    c0 = _norm3_tc(_cross_tc(ua, ub))
    c1 = _norm3_tc(_cross_tc(ub, ud))
    c2 = _norm3_tc(_cross_tc(ud, uc))
    c3 = _norm3_tc(_cross_tc(uc, ua))
    omega = (_asin_tc(jnp.clip(_dot3_tc(c0, c1), -1.0, 1.0))
             + _asin_tc(jnp.clip(_dot3_tc(c1, c2), -1.0, 1.0))
             + _asin_tc(jnp.clip(_dot3_tc(c2, c3), -1.0, 1.0))
             + _asin_tc(jnp.clip(_dot3_tc(c3, c0), -1.0, 1.0)))

    tj = tuple(_shl_lane(rows[c]) - rows[c] for c in range(3))
    ti = tuple(_shl_sub(cols[c]) - cols[c] for c in range(3))
    sgn = jnp.sign(_dot3_tc(_cross_tc(tj, ti), ua))
    wr = omega * sgn * jnp.float32(_INV2PI)

    ii = lax.broadcasted_iota(jnp.int32, (128, 128), 0)
    jj = lax.broadcasted_iota(jnp.int32, (128, 128), 1)
    mask = (ii + 2 <= jj) & (jj <= 98)
    wrs = jnp.where(mask, wr * jnp.float32(_ISTEP), 0.0)
    w1 = jnp.where(mask, w, 0.0)
    w2 = jnp.where(mask, _shl_sub(_shl_lane(w)), 0.0)

    s1 = jnp.sum(w1, axis=0, keepdims=True)
    s2 = jnp.sum(w2, axis=0, keepdims=True)
    denom = s1 + _shr_lane_row(s2)
    dinv = jnp.where(denom > 0, jnp.float32(1.0 / 1.12) / denom, 0.0)

    for k in range(_BINS):
        dk = wrs - jnp.float32(_CKS[k])
        e = jnp.exp(-(dk * dk))
        r1 = jnp.sum(w1 * e, axis=0, keepdims=True)
        r2v = jnp.sum(w2 * e, axis=0, keepdims=True)
        gt_ref[k:k + 1, :] = r1 + _shr_lane_row(r2v)

    gt = gt_ref[...] * dinv
    msg = lax.dot_general(gt, basis_ref[...], (((0,), (0,)), ((), ())),
                          preferred_element_type=jnp.float32)
    out_ref[0] = msg[:_N_ATOMS, :]


def _tc_finish_body(g_ref, basis_ref, out_ref):
    for i in range(_FB):
        g = g_ref[i]
        den = g[:, 64:65]
        dinv = jnp.where(den > 0, jnp.float32(1.0 / 1.12) / den, 0.0)
        gs = g[:, :_BINS] * dinv
        msg = lax.dot_general(gs, basis_ref[...], (((1,), (0,)), ((), ())),
                              preferred_element_type=jnp.float32)
        out_ref[i] = msg[:_N_ATOMS, :]


@jax.jit
def kernel(x, invariant_node_features, basis):
    xr3 = x.reshape(_BATCH, _N_ATOMS, 3)
    basis2 = basis[0, 0]

    xpad = jnp.pad(xr3[:_F_SC], ((0, 0), (0, 28), (0, 0)))
    xx, xy, xz = xpad[:, :, 0], xpad[:, :, 1], xpad[:, :, 2]
    zg = jnp.zeros((_GROWS, _GCOLS), jnp.float32)
    g = _sc_stage(jnp.asarray(_SEG_I), jnp.asarray(_SEG_J), xx, xy, xz, zg)

    xt = jnp.transpose(xr3[_F_SC:], (0, 2, 1))
    xt = jnp.pad(xt, ((0, 0), (0, 0), (0, 128 - _N_ATOMS)))
    msg_tc = pl.pallas_call(
        _dense_body,
        grid=(_F_TC,),
        in_specs=[
            pl.BlockSpec((1, 3, 128), lambda b: (b, 0, 0)),
            pl.BlockSpec((_BINS, _N_FEATURES), lambda b: (0, 0)),
        ],
        out_specs=pl.BlockSpec((1, _N_ATOMS, _N_FEATURES), lambda b: (b, 0, 0)),
        out_shape=jax.ShapeDtypeStruct((_F_TC, _N_ATOMS, _N_FEATURES), jnp.float32),
        scratch_shapes=[pltpu.VMEM((_BINS, 128), jnp.float32)],
    )(xt, basis2)

    msg_sc = pl.pallas_call(
        _tc_finish_body,
        grid=(_F_SC // _FB,),
        in_specs=[
            pl.BlockSpec((_FB, _GROWS, _GCOLS), lambda b: (b, 0, 0)),
            pl.BlockSpec((_BINS, _N_FEATURES), lambda b: (0, 0)),
        ],
        out_specs=pl.BlockSpec((_FB, _N_ATOMS, _N_FEATURES), lambda b: (b, 0, 0)),
        out_shape=jax.ShapeDtypeStruct((_F_SC, _N_ATOMS, _N_FEATURES), jnp.float32),
    )(g, basis2)

    msg = jnp.concatenate([msg_sc, msg_tc], axis=0).reshape(
        _BATCH * _N_ATOMS, _N_FEATURES)
    return invariant_node_features + msg


# speedup vs baseline: 2.2593x; 1.1908x over previous
import functools
import math
from collections import defaultdict
from itertools import combinations

import numpy as np
import jax
import jax.numpy as jnp
from jax import lax
from jax.experimental import pallas as pl
from jax.experimental.pallas import tpu as pltpu
from jax.experimental.pallas import tpu_sc as plsc

_N_ATOMS = 100
_N_FEATURES = 64
_BATCH = 128
_BINS = 64
_STEP = 2.0 / (_BINS - 1)
_ISTEP = 1.0 / _STEP
_CKS = [(-1.0 + k * _STEP) * _ISTEP for k in range(_BINS)]
_INV2PI = 1.0 / (2.0 * math.pi)

_NW = 32
_F_SC = 64
_F_TC = _BATCH - _F_SC
_GROWS = 120
_GCOLS = 66
_WIN = 9
_FB = 8


def _schedule_segments():
    pairs = np.array(list(combinations(range(_N_ATOMS - 1), 2)), dtype=np.int64)
    pairs = pairs[pairs[:, 1] - pairs[:, 0] != 1]
    buckets = defaultdict(list)
    for idx, (_, j) in enumerate(pairs):
        buckets[j].append(idx)
    chunks = []
    while buckets:
        js = sorted(buckets, key=lambda j: -len(buckets[j]))[:16]
        chunks.append([buckets[j].pop() for j in js])
        for j in js:
            if not buckets[j]:
                del buckets[j]
    si, sj = [], []
    for chunk in chunks:
        ci = [int(pairs[s, 0]) for s in chunk]
        cj = [int(pairs[s, 1]) for s in chunk]
        for lane in range(len(chunk), 16):
            ci.append(0)
            cj.append(100 + lane)
        si.extend(ci)
        sj.extend(cj)
    return (np.asarray(si, np.int32), np.asarray(sj, np.int32), len(chunks))


_SEG_I, _SEG_J, _NCHUNK = _schedule_segments()


def _rsqrt16(v):
    y = plsc.bitcast(jnp.int32(0x5F3759DF) - (plsc.bitcast(v, jnp.int32) >> 1),
                     jnp.float32)
    for _ in range(3):
        y = y * (1.5 - 0.5 * v * y * y)
    return y


def _asin16(x):
    ax = jnp.abs(x)
    p = jnp.full((16,), -0.0012624911, jnp.float32)
    for c in (0.0066700901, -0.0170881256, 0.0308918810,
              -0.0501743046, 0.0889789874, -0.2145988016, 1.5707963050):
        p = p * ax + jnp.float32(c)
    t = jnp.maximum(1.0 - ax, 0.0)
    s = t * _rsqrt16(t)
    return jnp.sign(x) * (jnp.float32(1.5707963267948966) - s * p)


def _cross(a, b):
    return (a[1] * b[2] - a[2] * b[1],
            a[2] * b[0] - a[0] * b[2],
            a[0] * b[1] - a[1] * b[0])


def _dot(a, b):
    return a[0] * b[0] + a[1] * b[1] + a[2] * b[2]


def _normed(v):
    r = _rsqrt16(_dot(v, v))
    return (v[0] * r, v[1] * r, v[2] * r)


def _sc_stage(segi, segj, xx, xy, xz, zg):
    mesh = plsc.VectorSubcoreMesh(core_axis_name="c", subcore_axis_name="s")

    @functools.partial(
        pl.kernel, mesh=mesh,
        compiler_params=pltpu.CompilerParams(needs_layout_passes=False),
        out_type=jax.ShapeDtypeStruct((_F_SC, _GROWS, _GCOLS), jnp.float32),
        scratch_types=[
            pltpu.VMEM((_NCHUNK * 16,), jnp.int32),
            pltpu.VMEM((_NCHUNK * 16,), jnp.int32),
            pltpu.VMEM((128,), jnp.float32),
            pltpu.VMEM((128,), jnp.float32),
            pltpu.VMEM((128,), jnp.float32),
            pltpu.VMEM((_GROWS, _GCOLS), jnp.float32),
        ],
    )
    def sc_g(segi_hbm, segj_hbm, xx_hbm, xy_hbm, xz_hbm, zg_hbm, g_hbm,
             segi_v, segj_v, xxv, xyv, xzv, gv):
        wid = lax.axis_index("s") * 2 + lax.axis_index("c")
        pltpu.sync_copy(segi_hbm, segi_v)
        pltpu.sync_copy(segj_hbm, segj_v)
        nf = _F_SC // _NW

        def frame_body(fi, carry):
            f = wid * nf + fi
            pltpu.sync_copy(xx_hbm.at[f], xxv)
            pltpu.sync_copy(xy_hbm.at[f], xyv)
            pltpu.sync_copy(xz_hbm.at[f], xzv)
            pltpu.sync_copy(zg_hbm, gv)

            def chunk_body(c, carry2):
                base = c * 16
                ii = segi_v[pl.ds(base, 16)]
                jj = segj_v[pl.ds(base, 16)]
                i1 = ii + 1
                j1 = jj + 1
                pi = tuple(plsc.load_gather(r, [ii]) for r in (xxv, xyv, xzv))
                pi1 = tuple(plsc.load_gather(r, [i1]) for r in (xxv, xyv, xzv))
                pj = tuple(plsc.load_gather(r, [jj]) for r in (xxv, xyv, xzv))
                pj1 = tuple(plsc.load_gather(r, [j1]) for r in (xxv, xyv, xzv))

                dx0 = tuple(pj[a] - pi[a] for a in range(3))
                dx1 = tuple(pj1[a] - pi[a] for a in range(3))
                dx2 = tuple(pj[a] - pi1[a] for a in range(3))
                dx3 = tuple(pj1[a] - pi1[a] for a in range(3))
                r2a = _dot(dx0, dx0)
                r2b = _dot(dx3, dx3)
                w1 = jnp.exp(-r2a)
                w2 = jnp.exp(-r2b)

                ra = _rsqrt16(r2a)
                rb = _rsqrt16(r2b)
                u0 = tuple(dx0[a] * ra for a in range(3))
                u3 = tuple(dx3[a] * rb for a in range(3))
                u1 = _normed(dx1)
                u2 = _normed(dx2)

                c0 = _normed(_cross(u0, u1))
                c1 = _normed(_cross(u1, u3))
                c2 = _normed(_cross(u3, u2))
                c3 = _normed(_cross(u2, u0))
                omega = (_asin16(jnp.clip(_dot(c0, c1), -1.0, 1.0))
                         + _asin16(jnp.clip(_dot(c1, c2), -1.0, 1.0))
                         + _asin16(jnp.clip(_dot(c2, c3), -1.0, 1.0))
                         + _asin16(jnp.clip(_dot(c3, c0), -1.0, 1.0)))
                tj = tuple(pj1[a] - pj[a] for a in range(3))
                ti = tuple(pi1[a] - pi[a] for a in range(3))
                sgn = jnp.sign(_dot(_cross(tj, ti), u0))
                vb = omega * sgn * jnp.float32(_INV2PI * _ISTEP) + jnp.float32(31.5)
                k0 = lax.convert_element_type(vb + 0.5, jnp.int32)

                col64 = jnp.full((16,), 64, jnp.int32)
                plsc.addupdate_scatter(gv, [jj, col64], w1)
                plsc.addupdate_scatter(gv, [j1, col64], w2)
                klo = k0 - _WIN // 2
                d0 = vb - lax.convert_element_type(klo, jnp.float32)
                e = jnp.exp(-(d0 * d0))
                ratio = jnp.exp(2.0 * d0 - 1.0)
                decay = jnp.full((16,), math.exp(-2.0), jnp.float32)
                for kk in range(_WIN):
                    k = klo + kk
                    valid = (k >= 0) & (k <= 63)
                    plsc.addupdate_scatter(gv, [jj, k], w1 * e, mask=valid)
                    plsc.addupdate_scatter(gv, [j1, k], w2 * e, mask=valid)
                    if kk < _WIN - 1:
                        e = e * ratio
                        ratio = ratio * decay
                return carry2

            lax.fori_loop(0, _NCHUNK, chunk_body, 0)
            pltpu.sync_copy(gv, g_hbm.at[f])
            return carry

        lax.fori_loop(0, nf, frame_body, 0)

    return sc_g(segi, segj, xx, xy, xz, zg)


def _shl_lane(m):
    return jnp.concatenate([m[:, 1:], m[:, :1]], axis=1)


def _shl_sub(m):
    return jnp.concatenate([m[1:, :], m[:1, :]], axis=0)


def _shr_lane_row(v):
    return jnp.concatenate([jnp.zeros((1, 1), jnp.float32), v[:, :-1]], axis=1)


def _asin_tc(x):
    ax = jnp.abs(x)
    p = jnp.float32(-0.0012624911)
    for c in (0.0066700901, -0.0170881256, 0.0308918810,
              -0.0501743046, 0.0889789874, -0.2145988016, 1.5707963050):
        p = p * ax + jnp.float32(c)
    r = jnp.float32(1.5707963267948966) - jnp.sqrt(jnp.maximum(1.0 - ax, 0.0)) * p
    return jnp.sign(x) * r


def _cross_tc(a, b):
    return (a[1] * b[2] - a[2] * b[1],
            a[2] * b[0] - a[0] * b[2],
            a[0] * b[1] - a[1] * b[0])


def _norm3_tc(v):
    r = lax.rsqrt(v[0] * v[0] + v[1] * v[1] + v[2] * v[2])
    return (v[0] * r, v[1] * r, v[2] * r)


def _dot3_tc(a, b):
    return a[0] * b[0] + a[1] * b[1] + a[2] * b[2]


_NR = 104


def _dense_body(xt_ref, basis_ref, out_ref, gt_ref):
    for fr in range(2):
        msg = _dense_frame(xt_ref[fr], basis_ref, gt_ref)
        out_ref[fr * _N_ATOMS:(fr + 1) * _N_ATOMS, :] = msg[:_N_ATOMS, :]


def _dense_frame(xr, basis_ref, gt_ref):
    cols = [jnp.transpose(jnp.broadcast_to(xr[c:c + 1, :], (128, 128)))[:_NR, :]
            for c in range(3)]
    rows = [jnp.broadcast_to(xr[c:c + 1, :], (_NR, 128)) for c in range(3)]
    d = [rows[c] - cols[c] for c in range(3)]
    r2 = d[0] * d[0] + d[1] * d[1] + d[2] * d[2]
    w = jnp.exp(-r2)
    inv = lax.rsqrt(r2)
    ua = tuple(d[c] * inv for c in range(3))
    ub = tuple(_shl_lane(u) for u in ua)
    uc = tuple(_shl_sub(u) for u in ua)
    ud = tuple(_shl_sub(u) for u in ub)

    c0 = _norm3_tc(_cross_tc(ua, ub))
    c1 = _norm3_tc(_cross_tc(ub, ud))
    c2 = _norm3_tc(_cross_tc(ud, uc))
    c3 = _norm3_tc(_cross_tc(uc, ua))
    omega = (_asin_tc(jnp.clip(_dot3_tc(c0, c1), -1.0, 1.0))
             + _asin_tc(jnp.clip(_dot3_tc(c1, c2), -1.0, 1.0))
             + _asin_tc(jnp.clip(_dot3_tc(c2, c3), -1.0, 1.0))
             + _asin_tc(jnp.clip(_dot3_tc(c3, c0), -1.0, 1.0)))

    tj = tuple(_shl_lane(rows[c]) - rows[c] for c in range(3))
    ti = tuple(_shl_sub(cols[c]) - cols[c] for c in range(3))
    sgn = jnp.sign(_dot3_tc(_cross_tc(tj, ti), ua))
    wr = omega * sgn * jnp.float32(_INV2PI)

    ii = lax.broadcasted_iota(jnp.int32, (_NR, 128), 0)
    jj = lax.broadcasted_iota(jnp.int32, (_NR, 128), 1)
    mask = (ii + 2 <= jj) & (jj <= 98)
    wrs = jnp.where(mask, wr * jnp.float32(_ISTEP), 0.0)
    w1 = jnp.where(mask, w, 0.0)
    w2 = jnp.where(mask, _shl_sub(_shl_lane(w)), 0.0)

    s1 = jnp.sum(w1, axis=0, keepdims=True)
    s2 = jnp.sum(w2, axis=0, keepdims=True)
    denom = s1 + _shr_lane_row(s2)
    dinv = jnp.where(denom > 0, jnp.float32(1.0 / 1.12) / denom, 0.0)

    for k in range(_BINS):
        dk = wrs - jnp.float32(_CKS[k])
        e = jnp.exp(-(dk * dk))
        r1 = jnp.sum(w1 * e, axis=0, keepdims=True)
        r2v = jnp.sum(w2 * e, axis=0, keepdims=True)
        gt_ref[k:k + 1, :] = r1 + _shr_lane_row(r2v)

    gt = gt_ref[...] * dinv
    return lax.dot_general(gt, basis_ref[...], (((0,), (0,)), ((), ())),
                           preferred_element_type=jnp.float32)


def _tc_finish_body(g_ref, basis_ref, out_ref):
    for i in range(_FB):
        g = g_ref[i]
        den = g[:, 64:65]
        dinv = jnp.where(den > 0, jnp.float32(1.0 / 1.12) / den, 0.0)
        gs = g[:, :_BINS] * dinv
        msg = lax.dot_general(gs, basis_ref[...], (((1,), (0,)), ((), ())),
                              preferred_element_type=jnp.float32)
        out_ref[i * _N_ATOMS:(i + 1) * _N_ATOMS, :] = msg[:_N_ATOMS, :]


@jax.jit
def kernel(x, invariant_node_features, basis):
    xr3 = x.reshape(_BATCH, _N_ATOMS, 3)
    basis2 = basis[0, 0]

    xpad = jnp.pad(xr3[:_F_SC], ((0, 0), (0, 28), (0, 0)))
    xx, xy, xz = xpad[:, :, 0], xpad[:, :, 1], xpad[:, :, 2]
    zg = jnp.zeros((_GROWS, _GCOLS), jnp.float32)
    g = _sc_stage(jnp.asarray(_SEG_I), jnp.asarray(_SEG_J), xx, xy, xz, zg)

    xt = jnp.transpose(xr3[_F_SC:], (0, 2, 1))
    xt = jnp.pad(xt, ((0, 0), (0, 0), (0, 128 - _N_ATOMS)))
    msg_tc = pl.pallas_call(
        _dense_body,
        grid=(_F_TC // 2,),
        in_specs=[
            pl.BlockSpec((2, 3, 128), lambda b: (b, 0, 0)),
            pl.BlockSpec((_BINS, _N_FEATURES), lambda b: (0, 0)),
        ],
        out_specs=pl.BlockSpec((2 * _N_ATOMS, _N_FEATURES), lambda b: (b, 0)),
        out_shape=jax.ShapeDtypeStruct((_F_TC * _N_ATOMS, _N_FEATURES),
                                       jnp.float32),
        scratch_shapes=[pltpu.VMEM((_BINS, 128), jnp.float32)],
    )(xt, basis2)

    msg_sc = pl.pallas_call(
        _tc_finish_body,
        grid=(_F_SC // _FB,),
        in_specs=[
            pl.BlockSpec((_FB, _GROWS, _GCOLS), lambda b: (b, 0, 0)),
            pl.BlockSpec((_BINS, _N_FEATURES), lambda b: (0, 0)),
        ],
        out_specs=pl.BlockSpec((_FB * _N_ATOMS, _N_FEATURES), lambda b: (b, 0)),
        out_shape=jax.ShapeDtypeStruct((_F_SC * _N_ATOMS, _N_FEATURES),
                                       jnp.float32),
    )(g, basis2)

    msg = jnp.concatenate([msg_sc, msg_tc], axis=0)
    return invariant_node_features + msg


# speedup vs baseline: 2.3118x; 1.0232x over previous
import functools
import math
from collections import defaultdict
from itertools import combinations

import numpy as np
import jax
import jax.numpy as jnp
from jax import lax
from jax.experimental import pallas as pl
from jax.experimental.pallas import tpu as pltpu
from jax.experimental.pallas import tpu_sc as plsc

_N_ATOMS = 100
_N_FEATURES = 64
_BATCH = 128
_BINS = 64
_STEP = 2.0 / (_BINS - 1)
_ISTEP = 1.0 / _STEP
_CKS = [(-1.0 + k * _STEP) * _ISTEP for k in range(_BINS)]
_INV2PI = 1.0 / (2.0 * math.pi)

_NW = 32
_F_SC = 64
_F_TC = _BATCH - _F_SC
_GROWS = 120
_GCOLS = 66
_WIN = 7
_FB = 8


def _schedule_segments():
    pairs = np.array(list(combinations(range(_N_ATOMS - 1), 2)), dtype=np.int64)
    pairs = pairs[pairs[:, 1] - pairs[:, 0] != 1]
    buckets = defaultdict(list)
    for idx, (_, j) in enumerate(pairs):
        buckets[j].append(idx)
    chunks = []
    while buckets:
        js = sorted(buckets, key=lambda j: -len(buckets[j]))[:16]
        chunks.append([buckets[j].pop() for j in js])
        for j in js:
            if not buckets[j]:
                del buckets[j]
    si, sj = [], []
    for chunk in chunks:
        ci = [int(pairs[s, 0]) for s in chunk]
        cj = [int(pairs[s, 1]) for s in chunk]
        for lane in range(len(chunk), 16):
            ci.append(0)
            cj.append(100 + lane)
        si.extend(ci)
        sj.extend(cj)
    return (np.asarray(si, np.int32), np.asarray(sj, np.int32), len(chunks))


_SEG_I, _SEG_J, _NCHUNK = _schedule_segments()


def _rsqrt16(v):
    y = plsc.bitcast(jnp.int32(0x5F3759DF) - (plsc.bitcast(v, jnp.int32) >> 1),
                     jnp.float32)
    for _ in range(2):
        y = y * (1.5 - 0.5 * v * y * y)
    return y


def _asin16(x):
    ax = jnp.abs(x)
    p = jnp.full((16,), -0.0012624911, jnp.float32)
    for c in (0.0066700901, -0.0170881256, 0.0308918810,
              -0.0501743046, 0.0889789874, -0.2145988016, 1.5707963050):
        p = p * ax + jnp.float32(c)
    t = jnp.maximum(1.0 - ax, 0.0)
    s = t * _rsqrt16(t)
    return jnp.sign(x) * (jnp.float32(1.5707963267948966) - s * p)


def _cross(a, b):
    return (a[1] * b[2] - a[2] * b[1],
            a[2] * b[0] - a[0] * b[2],
            a[0] * b[1] - a[1] * b[0])


def _dot(a, b):
    return a[0] * b[0] + a[1] * b[1] + a[2] * b[2]


def _normed(v):
    r = _rsqrt16(_dot(v, v))
    return (v[0] * r, v[1] * r, v[2] * r)


def _sc_stage(segi, segj, xx, xy, xz, zg):
    mesh = plsc.VectorSubcoreMesh(core_axis_name="c", subcore_axis_name="s")

    @functools.partial(
        pl.kernel, mesh=mesh,
        compiler_params=pltpu.CompilerParams(needs_layout_passes=False),
        out_type=jax.ShapeDtypeStruct((_F_SC, _GROWS, _GCOLS), jnp.float32),
        scratch_types=[
            pltpu.VMEM((_NCHUNK * 16,), jnp.int32),
            pltpu.VMEM((_NCHUNK * 16,), jnp.int32),
            pltpu.VMEM((128,), jnp.float32),
            pltpu.VMEM((128,), jnp.float32),
            pltpu.VMEM((128,), jnp.float32),
            pltpu.VMEM((_GROWS, _GCOLS), jnp.float32),
        ],
    )
    def sc_g(segi_hbm, segj_hbm, xx_hbm, xy_hbm, xz_hbm, zg_hbm, g_hbm,
             segi_v, segj_v, xxv, xyv, xzv, gv):
        wid = lax.axis_index("s") * 2 + lax.axis_index("c")
        pltpu.sync_copy(segi_hbm, segi_v)
        pltpu.sync_copy(segj_hbm, segj_v)
        nf = _F_SC // _NW

        def frame_body(fi, carry):
            f = wid * nf + fi
            pltpu.sync_copy(xx_hbm.at[f], xxv)
            pltpu.sync_copy(xy_hbm.at[f], xyv)
            pltpu.sync_copy(xz_hbm.at[f], xzv)
            pltpu.sync_copy(zg_hbm, gv)

            def chunk_body(c, carry2):
                base = c * 16
                ii = segi_v[pl.ds(base, 16)]
                jj = segj_v[pl.ds(base, 16)]
                i1 = ii + 1
                j1 = jj + 1
                pi = tuple(plsc.load_gather(r, [ii]) for r in (xxv, xyv, xzv))
                pi1 = tuple(plsc.load_gather(r, [i1]) for r in (xxv, xyv, xzv))
                pj = tuple(plsc.load_gather(r, [jj]) for r in (xxv, xyv, xzv))
                pj1 = tuple(plsc.load_gather(r, [j1]) for r in (xxv, xyv, xzv))

                dx0 = tuple(pj[a] - pi[a] for a in range(3))
                dx1 = tuple(pj1[a] - pi[a] for a in range(3))
                dx2 = tuple(pj[a] - pi1[a] for a in range(3))
                dx3 = tuple(pj1[a] - pi1[a] for a in range(3))
                r2a = _dot(dx0, dx0)
                r2b = _dot(dx3, dx3)
                w1 = jnp.exp(-r2a)
                w2 = jnp.exp(-r2b)

                ra = _rsqrt16(r2a)
                rb = _rsqrt16(r2b)
                u0 = tuple(dx0[a] * ra for a in range(3))
                u3 = tuple(dx3[a] * rb for a in range(3))
                u1 = _normed(dx1)
                u2 = _normed(dx2)

                c0 = _normed(_cross(u0, u1))
                c1 = _normed(_cross(u1, u3))
                c2 = _normed(_cross(u3, u2))
                c3 = _normed(_cross(u2, u0))
                omega = (_asin16(jnp.clip(_dot(c0, c1), -1.0, 1.0))
                         + _asin16(jnp.clip(_dot(c1, c2), -1.0, 1.0))
                         + _asin16(jnp.clip(_dot(c2, c3), -1.0, 1.0))
                         + _asin16(jnp.clip(_dot(c3, c0), -1.0, 1.0)))
                tj = tuple(pj1[a] - pj[a] for a in range(3))
                ti = tuple(pi1[a] - pi[a] for a in range(3))
                sgn = jnp.sign(_dot(_cross(tj, ti), u0))
                vb = omega * sgn * jnp.float32(_INV2PI * _ISTEP) + jnp.float32(31.5)
                k0 = lax.convert_element_type(vb + 0.5, jnp.int32)

                col64 = jnp.full((16,), 64, jnp.int32)
                plsc.addupdate_scatter(gv, [jj, col64], w1)
                plsc.addupdate_scatter(gv, [j1, col64], w2)
                klo = k0 - _WIN // 2
                d0 = vb - lax.convert_element_type(klo, jnp.float32)
                e = jnp.exp(-(d0 * d0))
                ratio = jnp.exp(2.0 * d0 - 1.0)
                decay = jnp.full((16,), math.exp(-2.0), jnp.float32)
                for kk in range(_WIN):
                    k = klo + kk
                    valid = (k >= 0) & (k <= 63)
                    plsc.addupdate_scatter(gv, [jj, k], w1 * e, mask=valid)
                    plsc.addupdate_scatter(gv, [j1, k], w2 * e, mask=valid)
                    if kk < _WIN - 1:
                        e = e * ratio
                        ratio = ratio * decay
                return carry2

            lax.fori_loop(0, _NCHUNK, chunk_body, 0)
            pltpu.sync_copy(gv, g_hbm.at[f])
            return carry

        lax.fori_loop(0, nf, frame_body, 0)

    return sc_g(segi, segj, xx, xy, xz, zg)


def _shl_lane(m):
    return jnp.concatenate([m[:, 1:], m[:, :1]], axis=1)


def _shl_sub(m):
    return jnp.concatenate([m[1:, :], m[:1, :]], axis=0)


def _shr_lane_row(v):
    return jnp.concatenate([jnp.zeros((1, 1), jnp.float32), v[:, :-1]], axis=1)


def _asin_tc(x):
    ax = jnp.abs(x)
    p = jnp.float32(-0.0012624911)
    for c in (0.0066700901, -0.0170881256, 0.0308918810,
              -0.0501743046, 0.0889789874, -0.2145988016, 1.5707963050):
        p = p * ax + jnp.float32(c)
    r = jnp.float32(1.5707963267948966) - jnp.sqrt(jnp.maximum(1.0 - ax, 0.0)) * p
    return jnp.sign(x) * r


def _cross_tc(a, b):
    return (a[1] * b[2] - a[2] * b[1],
            a[2] * b[0] - a[0] * b[2],
            a[0] * b[1] - a[1] * b[0])


def _norm3_tc(v):
    r = lax.rsqrt(v[0] * v[0] + v[1] * v[1] + v[2] * v[2])
    return (v[0] * r, v[1] * r, v[2] * r)


def _dot3_tc(a, b):
    return a[0] * b[0] + a[1] * b[1] + a[2] * b[2]


_NR = 104


def _dense_body(xt_ref, nodef_ref, basis_ref, out_ref, gt_ref):
    for fr in range(2):
        msg = _dense_frame(xt_ref[fr], basis_ref, gt_ref)
        sl = slice(fr * _N_ATOMS, (fr + 1) * _N_ATOMS)
        out_ref[sl, :] = nodef_ref[sl, :] + msg[:_N_ATOMS, :]


def _dense_frame(xr, basis_ref, gt_ref):
    cols = [jnp.transpose(jnp.broadcast_to(xr[c:c + 1, :], (128, 128)))[:_NR, :]
            for c in range(3)]
    rows = [jnp.broadcast_to(xr[c:c + 1, :], (_NR, 128)) for c in range(3)]
    d = [rows[c] - cols[c] for c in range(3)]
    r2 = d[0] * d[0] + d[1] * d[1] + d[2] * d[2]
    w = jnp.exp(-r2)
    inv = lax.rsqrt(r2)
    ua = tuple(d[c] * inv for c in range(3))
    ub = tuple(_shl_lane(u) for u in ua)
    uc = tuple(_shl_sub(u) for u in ua)
    ud = tuple(_shl_sub(u) for u in ub)

    c0 = _norm3_tc(_cross_tc(ua, ub))
    c1 = _norm3_tc(_cross_tc(ub, ud))
    c2 = _norm3_tc(_cross_tc(ud, uc))
    c3 = _norm3_tc(_cross_tc(uc, ua))
    omega = (_asin_tc(jnp.clip(_dot3_tc(c0, c1), -1.0, 1.0))
             + _asin_tc(jnp.clip(_dot3_tc(c1, c2), -1.0, 1.0))
             + _asin_tc(jnp.clip(_dot3_tc(c2, c3), -1.0, 1.0))
             + _asin_tc(jnp.clip(_dot3_tc(c3, c0), -1.0, 1.0)))

    tj = tuple(_shl_lane(rows[c]) - rows[c] for c in range(3))
    ti = tuple(_shl_sub(cols[c]) - cols[c] for c in range(3))
    sgn = jnp.sign(_dot3_tc(_cross_tc(tj, ti), ua))
    wr = omega * sgn * jnp.float32(_INV2PI)

    ii = lax.broadcasted_iota(jnp.int32, (_NR, 128), 0)
    jj = lax.broadcasted_iota(jnp.int32, (_NR, 128), 1)
    mask = (ii + 2 <= jj) & (jj <= 98)
    wrs = jnp.where(mask, wr * jnp.float32(_ISTEP), 0.0)
    w1 = jnp.where(mask, w, 0.0)
    w2 = jnp.where(mask, _shl_sub(_shl_lane(w)), 0.0)

    s1 = jnp.sum(w1, axis=0, keepdims=True)
    s2 = jnp.sum(w2, axis=0, keepdims=True)
    denom = s1 + _shr_lane_row(s2)
    dinv = jnp.where(denom > 0, jnp.float32(1.0 / 1.12) / denom, 0.0)

    for k in range(_BINS):
        dk = wrs - jnp.float32(_CKS[k])
        e = jnp.exp(-(dk * dk))
        r1 = jnp.sum(w1 * e, axis=0, keepdims=True)
        r2v = jnp.sum(w2 * e, axis=0, keepdims=True)
        gt_ref[k:k + 1, :] = r1 + _shr_lane_row(r2v)

    gt = gt_ref[...] * dinv
    return lax.dot_general(gt, basis_ref[...], (((0,), (0,)), ((), ())),
                           preferred_element_type=jnp.float32)


def _tc_finish_body(g_ref, nodef_ref, basis_ref, out_ref):
    for i in range(_FB):
        g = g_ref[i]
        den = g[:, 64:65]
        dinv = jnp.where(den > 0, jnp.float32(1.0 / 1.12) / den, 0.0)
        gs = g[:, :_BINS] * dinv
        msg = lax.dot_general(gs, basis_ref[...], (((1,), (0,)), ((), ())),
                              preferred_element_type=jnp.float32)
        sl = slice(i * _N_ATOMS, (i + 1) * _N_ATOMS)
        out_ref[sl, :] = nodef_ref[sl, :] + msg[:_N_ATOMS, :]


@jax.jit
def kernel(x, invariant_node_features, basis):
    xr3 = x.reshape(_BATCH, _N_ATOMS, 3)
    basis2 = basis[0, 0]

    xpad = jnp.pad(xr3[:_F_SC], ((0, 0), (0, 28), (0, 0)))
    xx, xy, xz = xpad[:, :, 0], xpad[:, :, 1], xpad[:, :, 2]
    zg = jnp.zeros((_GROWS, _GCOLS), jnp.float32)
    g = _sc_stage(jnp.asarray(_SEG_I), jnp.asarray(_SEG_J), xx, xy, xz, zg)

    xt = jnp.transpose(xr3[_F_SC:], (0, 2, 1))
    xt = jnp.pad(xt, ((0, 0), (0, 0), (0, 128 - _N_ATOMS)))
    msg_tc = pl.pallas_call(
        _dense_body,
        grid=(_F_TC // 2,),
        in_specs=[
            pl.BlockSpec((2, 3, 128), lambda b: (b, 0, 0)),
            pl.BlockSpec((2 * _N_ATOMS, _N_FEATURES),
                         lambda b: (b + _F_SC // 2, 0)),
            pl.BlockSpec((_BINS, _N_FEATURES), lambda b: (0, 0)),
        ],
        out_specs=pl.BlockSpec((2 * _N_ATOMS, _N_FEATURES), lambda b: (b, 0)),
        out_shape=jax.ShapeDtypeStruct((_F_TC * _N_ATOMS, _N_FEATURES),
                                       jnp.float32),
        scratch_shapes=[pltpu.VMEM((_BINS, 128), jnp.float32)],
    )(xt, invariant_node_features, basis2)

    msg_sc = pl.pallas_call(
        _tc_finish_body,
        grid=(_F_SC // _FB,),
        in_specs=[
            pl.BlockSpec((_FB, _GROWS, _GCOLS), lambda b: (b, 0, 0)),
            pl.BlockSpec((_FB * _N_ATOMS, _N_FEATURES), lambda b: (b, 0)),
            pl.BlockSpec((_BINS, _N_FEATURES), lambda b: (0, 0)),
        ],
        out_specs=pl.BlockSpec((_FB * _N_ATOMS, _N_FEATURES), lambda b: (b, 0)),
        out_shape=jax.ShapeDtypeStruct((_F_SC * _N_ATOMS, _N_FEATURES),
                                       jnp.float32),
    )(g, invariant_node_features, basis2)

    return jnp.concatenate([msg_sc, msg_tc], axis=0)


# speedup vs baseline: 2.3215x; 1.0042x over previous
import functools
import math
from collections import defaultdict
from itertools import combinations

import numpy as np
import jax
import jax.numpy as jnp
from jax import lax
from jax.experimental import pallas as pl
from jax.experimental.pallas import tpu as pltpu
from jax.experimental.pallas import tpu_sc as plsc

_N_ATOMS = 100
_N_FEATURES = 64
_BATCH = 128
_BINS = 64
_STEP = 2.0 / (_BINS - 1)
_ISTEP = 1.0 / _STEP
_CKS = [(-1.0 + k * _STEP) * _ISTEP for k in range(_BINS)]
_INV2PI = 1.0 / (2.0 * math.pi)

_NW = 32
_F_SC = 64
_F_TC = _BATCH - _F_SC
_GROWS = 120
_GCOLS = 66
_WIN = 7
_FB = 8


def _schedule_segments():
    pairs = np.array(list(combinations(range(_N_ATOMS - 1), 2)), dtype=np.int64)
    pairs = pairs[pairs[:, 1] - pairs[:, 0] != 1]
    buckets = defaultdict(list)
    for idx, (_, j) in enumerate(pairs):
        buckets[j].append(idx)
    chunks = []
    while buckets:
        js = sorted(buckets, key=lambda j: -len(buckets[j]))[:16]
        chunks.append([buckets[j].pop() for j in js])
        for j in js:
            if not buckets[j]:
                del buckets[j]
    si, sj = [], []
    for chunk in chunks:
        ci = [int(pairs[s, 0]) for s in chunk]
        cj = [int(pairs[s, 1]) for s in chunk]
        for lane in range(len(chunk), 16):
            ci.append(0)
            cj.append(100 + lane)
        si.extend(ci)
        sj.extend(cj)
    return (np.asarray(si, np.int32), np.asarray(sj, np.int32), len(chunks))


_SEG_I, _SEG_J, _NCHUNK = _schedule_segments()


def _rsqrt16(v):
    y = plsc.bitcast(jnp.int32(0x5F3759DF) - (plsc.bitcast(v, jnp.int32) >> 1),
                     jnp.float32)
    for _ in range(2):
        y = y * (1.5 - 0.5 * v * y * y)
    return y


def _asin16(x):
    ax = jnp.abs(x)
    p = jnp.full((16,), -0.0012624911, jnp.float32)
    for c in (0.0066700901, -0.0170881256, 0.0308918810,
              -0.0501743046, 0.0889789874, -0.2145988016, 1.5707963050):
        p = p * ax + jnp.float32(c)
    t = jnp.maximum(1.0 - ax, 0.0)
    s = t * _rsqrt16(t)
    return jnp.sign(x) * (jnp.float32(1.5707963267948966) - s * p)


def _cross(a, b):
    return (a[1] * b[2] - a[2] * b[1],
            a[2] * b[0] - a[0] * b[2],
            a[0] * b[1] - a[1] * b[0])


def _dot(a, b):
    return a[0] * b[0] + a[1] * b[1] + a[2] * b[2]


def _normed(v):
    r = _rsqrt16(_dot(v, v))
    return (v[0] * r, v[1] * r, v[2] * r)


def _sc_stage(segi, segj, xx, xy, xz, zg):
    mesh = plsc.VectorSubcoreMesh(core_axis_name="c", subcore_axis_name="s")

    @functools.partial(
        pl.kernel, mesh=mesh,
        compiler_params=pltpu.CompilerParams(needs_layout_passes=False),
        out_type=jax.ShapeDtypeStruct((_F_SC, _GROWS, _GCOLS), jnp.float32),
        scratch_types=[
            pltpu.VMEM((_NCHUNK * 16,), jnp.int32),
            pltpu.VMEM((_NCHUNK * 16,), jnp.int32),
            pltpu.VMEM((128,), jnp.float32),
            pltpu.VMEM((128,), jnp.float32),
            pltpu.VMEM((128,), jnp.float32),
            pltpu.VMEM((_GROWS, _GCOLS), jnp.float32),
        ],
    )
    def sc_g(segi_hbm, segj_hbm, xx_hbm, xy_hbm, xz_hbm, zg_hbm, g_hbm,
             segi_v, segj_v, xxv, xyv, xzv, gv):
        wid = lax.axis_index("s") * 2 + lax.axis_index("c")
        pltpu.sync_copy(segi_hbm, segi_v)
        pltpu.sync_copy(segj_hbm, segj_v)
        nf = _F_SC // _NW

        def frame_body(fi, carry):
            f = wid * nf + fi
            pltpu.sync_copy(xx_hbm.at[f], xxv)
            pltpu.sync_copy(xy_hbm.at[f], xyv)
            pltpu.sync_copy(xz_hbm.at[f], xzv)
            pltpu.sync_copy(zg_hbm, gv)

            def chunk_body(c, carry2):
                base = c * 16
                ii = segi_v[pl.ds(base, 16)]
                jj = segj_v[pl.ds(base, 16)]
                i1 = ii + 1
                j1 = jj + 1
                pi = tuple(plsc.load_gather(r, [ii]) for r in (xxv, xyv, xzv))
                pi1 = tuple(plsc.load_gather(r, [i1]) for r in (xxv, xyv, xzv))
                pj = tuple(plsc.load_gather(r, [jj]) for r in (xxv, xyv, xzv))
                pj1 = tuple(plsc.load_gather(r, [j1]) for r in (xxv, xyv, xzv))

                dx0 = tuple(pj[a] - pi[a] for a in range(3))
                dx1 = tuple(pj1[a] - pi[a] for a in range(3))
                dx2 = tuple(pj[a] - pi1[a] for a in range(3))
                dx3 = tuple(pj1[a] - pi1[a] for a in range(3))
                r2a = _dot(dx0, dx0)
                r2b = _dot(dx3, dx3)
                w1 = jnp.exp(-r2a)
                w2 = jnp.exp(-r2b)

                ra = _rsqrt16(r2a)
                rb = _rsqrt16(r2b)
                u0 = tuple(dx0[a] * ra for a in range(3))
                u3 = tuple(dx3[a] * rb for a in range(3))
                u1 = _normed(dx1)
                u2 = _normed(dx2)

                c0 = _normed(_cross(u0, u1))
                c1 = _normed(_cross(u1, u3))
                c2 = _normed(_cross(u3, u2))
                c3 = _normed(_cross(u2, u0))
                omega = (_asin16(jnp.clip(_dot(c0, c1), -1.0, 1.0))
                         + _asin16(jnp.clip(_dot(c1, c2), -1.0, 1.0))
                         + _asin16(jnp.clip(_dot(c2, c3), -1.0, 1.0))
                         + _asin16(jnp.clip(_dot(c3, c0), -1.0, 1.0)))
                tj = tuple(pj1[a] - pj[a] for a in range(3))
                ti = tuple(pi1[a] - pi[a] for a in range(3))
                sgn = jnp.sign(_dot(_cross(tj, ti), u0))
                vb = omega * sgn * jnp.float32(_INV2PI * _ISTEP) + jnp.float32(31.5)
                k0 = lax.convert_element_type(vb + 0.5, jnp.int32)

                col64 = jnp.full((16,), 64, jnp.int32)
                plsc.addupdate_scatter(gv, [jj, col64], w1)
                plsc.addupdate_scatter(gv, [j1, col64], w2)
                klo = k0 - _WIN // 2
                d0 = vb - lax.convert_element_type(klo, jnp.float32)
                e = jnp.exp(-(d0 * d0))
                ratio = jnp.exp(2.0 * d0 - 1.0)
                decay = jnp.full((16,), math.exp(-2.0), jnp.float32)
                for kk in range(_WIN):
                    k = klo + kk
                    valid = (k >= 0) & (k <= 63)
                    plsc.addupdate_scatter(gv, [jj, k], w1 * e, mask=valid)
                    plsc.addupdate_scatter(gv, [j1, k], w2 * e, mask=valid)
                    if kk < _WIN - 1:
                        e = e * ratio
                        ratio = ratio * decay
                return carry2

            lax.fori_loop(0, _NCHUNK, chunk_body, 0)
            pltpu.sync_copy(gv, g_hbm.at[f])
            return carry

        lax.fori_loop(0, nf, frame_body, 0)

    return sc_g(segi, segj, xx, xy, xz, zg)


def _shl_lane(m):
    return jnp.concatenate([m[:, 1:], m[:, :1]], axis=1)


def _shl_sub(m):
    return jnp.concatenate([m[1:, :], m[:1, :]], axis=0)


def _shr_lane_row(v):
    return jnp.concatenate([jnp.zeros((1, 1), jnp.float32), v[:, :-1]], axis=1)


def _asin_tc(x):
    ax = jnp.abs(x)
    p = jnp.float32(-0.0012624911)
    for c in (0.0066700901, -0.0170881256, 0.0308918810,
              -0.0501743046, 0.0889789874, -0.2145988016, 1.5707963050):
        p = p * ax + jnp.float32(c)
    r = jnp.float32(1.5707963267948966) - jnp.sqrt(jnp.maximum(1.0 - ax, 0.0)) * p
    return jnp.sign(x) * r


def _cross_tc(a, b):
    return (a[1] * b[2] - a[2] * b[1],
            a[2] * b[0] - a[0] * b[2],
            a[0] * b[1] - a[1] * b[0])


def _norm3_tc(v):
    r = lax.rsqrt(v[0] * v[0] + v[1] * v[1] + v[2] * v[2])
    return (v[0] * r, v[1] * r, v[2] * r)


def _dot3_tc(a, b):
    return a[0] * b[0] + a[1] * b[1] + a[2] * b[2]


_NR = 104


def _dense_body(xt_ref, nodef_ref, basis_ref, out_ref, gt_ref):
    for fr in range(2):
        msg = _dense_frame(xt_ref[fr], basis_ref, gt_ref)
        sl = slice(fr * _N_ATOMS, (fr + 1) * _N_ATOMS)
        out_ref[sl, :] = nodef_ref[sl, :] + msg[:_N_ATOMS, :]


def _dense_frame(xr, basis_ref, gt_ref):
    cols = [jnp.transpose(jnp.broadcast_to(xr[c:c + 1, :], (128, 128)))[:_NR, :]
            for c in range(3)]
    rows = [jnp.broadcast_to(xr[c:c + 1, :], (_NR, 128)) for c in range(3)]
    d = [rows[c] - cols[c] for c in range(3)]
    r2 = d[0] * d[0] + d[1] * d[1] + d[2] * d[2]
    w = jnp.exp(-r2)
    inv = lax.rsqrt(r2)
    ua = tuple(d[c] * inv for c in range(3))
    ub = tuple(_shl_lane(u) for u in ua)
    uc = tuple(_shl_sub(u) for u in ua)
    ud = tuple(_shl_sub(u) for u in ub)

    c0 = _norm3_tc(_cross_tc(ua, ub))
    c1 = _norm3_tc(_cross_tc(ub, ud))
    c2 = _norm3_tc(_cross_tc(ud, uc))
    c3 = _norm3_tc(_cross_tc(uc, ua))
    omega = (_asin_tc(jnp.clip(_dot3_tc(c0, c1), -1.0, 1.0))
             + _asin_tc(jnp.clip(_dot3_tc(c1, c2), -1.0, 1.0))
             + _asin_tc(jnp.clip(_dot3_tc(c2, c3), -1.0, 1.0))
             + _asin_tc(jnp.clip(_dot3_tc(c3, c0), -1.0, 1.0)))

    tj = tuple(_shl_lane(rows[c]) - rows[c] for c in range(3))
    ti = tuple(_shl_sub(cols[c]) - cols[c] for c in range(3))
    sgn = jnp.sign(_dot3_tc(_cross_tc(tj, ti), ua))
    wr = omega * sgn * jnp.float32(_INV2PI)

    ii = lax.broadcasted_iota(jnp.int32, (_NR, 128), 0)
    jj = lax.broadcasted_iota(jnp.int32, (_NR, 128), 1)
    mask = (ii + 2 <= jj) & (jj <= 98)
    wrs = jnp.where(mask, wr * jnp.float32(_ISTEP), 0.0)
    w1 = jnp.where(mask, w, 0.0)
    w2 = jnp.where(mask, _shl_sub(_shl_lane(w)), 0.0)

    s1 = jnp.sum(w1, axis=0, keepdims=True)
    s2 = jnp.sum(w2, axis=0, keepdims=True)
    denom = s1 + _shr_lane_row(s2)
    dinv = jnp.where(denom > 0, jnp.float32(1.0 / 1.12) / denom, 0.0)

    onesr = jnp.ones((1, _NR), jnp.float32)
    cdims = (((1,), (0,)), ((), ()))
    for k in range(_BINS):
        dk = wrs - jnp.float32(_CKS[k])
        e = jnp.exp(-(dk * dk))
        r1 = lax.dot_general(onesr, w1 * e, cdims,
                             preferred_element_type=jnp.float32)
        r2v = lax.dot_general(onesr, w2 * e, cdims,
                              preferred_element_type=jnp.float32)
        gt_ref[k:k + 1, :] = r1 + _shr_lane_row(r2v)

    gt = gt_ref[...] * dinv
    return lax.dot_general(gt, basis_ref[...], (((0,), (0,)), ((), ())),
                           preferred_element_type=jnp.float32)


def _tc_finish_body(g_ref, nodef_ref, basis_ref, out_ref):
    for i in range(_FB):
        g = g_ref[i]
        den = g[:, 64:65]
        dinv = jnp.where(den > 0, jnp.float32(1.0 / 1.12) / den, 0.0)
        gs = g[:, :_BINS] * dinv
        msg = lax.dot_general(gs, basis_ref[...], (((1,), (0,)), ((), ())),
                              preferred_element_type=jnp.float32)
        sl = slice(i * _N_ATOMS, (i + 1) * _N_ATOMS)
        out_ref[sl, :] = nodef_ref[sl, :] + msg[:_N_ATOMS, :]


@jax.jit
def kernel(x, invariant_node_features, basis):
    xr3 = x.reshape(_BATCH, _N_ATOMS, 3)
    basis2 = basis[0, 0]

    xpad = jnp.pad(xr3[:_F_SC], ((0, 0), (0, 28), (0, 0)))
    xx, xy, xz = xpad[:, :, 0], xpad[:, :, 1], xpad[:, :, 2]
    zg = jnp.zeros((_GROWS, _GCOLS), jnp.float32)
    g = _sc_stage(jnp.asarray(_SEG_I), jnp.asarray(_SEG_J), xx, xy, xz, zg)

    xt = jnp.transpose(xr3[_F_SC:], (0, 2, 1))
    xt = jnp.pad(xt, ((0, 0), (0, 0), (0, 128 - _N_ATOMS)))
    msg_tc = pl.pallas_call(
        _dense_body,
        grid=(_F_TC // 2,),
        in_specs=[
            pl.BlockSpec((2, 3, 128), lambda b: (b, 0, 0)),
            pl.BlockSpec((2 * _N_ATOMS, _N_FEATURES),
                         lambda b: (b + _F_SC // 2, 0)),
            pl.BlockSpec((_BINS, _N_FEATURES), lambda b: (0, 0)),
        ],
        out_specs=pl.BlockSpec((2 * _N_ATOMS, _N_FEATURES), lambda b: (b, 0)),
        out_shape=jax.ShapeDtypeStruct((_F_TC * _N_ATOMS, _N_FEATURES),
                                       jnp.float32),
        scratch_shapes=[pltpu.VMEM((_BINS, 128), jnp.float32)],
    )(xt, invariant_node_features, basis2)

    msg_sc = pl.pallas_call(
        _tc_finish_body,
        grid=(_F_SC // _FB,),
        in_specs=[
            pl.BlockSpec((_FB, _GROWS, _GCOLS), lambda b: (b, 0, 0)),
            pl.BlockSpec((_FB * _N_ATOMS, _N_FEATURES), lambda b: (b, 0)),
            pl.BlockSpec((_BINS, _N_FEATURES), lambda b: (0, 0)),
        ],
        out_specs=pl.BlockSpec((_FB * _N_ATOMS, _N_FEATURES), lambda b: (b, 0)),
        out_shape=jax.ShapeDtypeStruct((_F_SC * _N_ATOMS, _N_FEATURES),
                                       jnp.float32),
    )(g, invariant_node_features, basis2)

    return jnp.concatenate([msg_sc, msg_tc], axis=0)


# speedup vs baseline: 2.4234x; 1.0439x over previous
import functools
import math
from collections import defaultdict
from itertools import combinations

import numpy as np
import jax
import jax.numpy as jnp
from jax import lax
from jax.experimental import pallas as pl
from jax.experimental.pallas import tpu as pltpu
from jax.experimental.pallas import tpu_sc as plsc

_N_ATOMS = 100
_N_FEATURES = 64
_BATCH = 128
_BINS = 64
_STEP = 2.0 / (_BINS - 1)
_ISTEP = 1.0 / _STEP
_CKS = [(-1.0 + k * _STEP) * _ISTEP for k in range(_BINS)]
_INV2PI = 1.0 / (2.0 * math.pi)

_NW = 32
_F_SC = 64
_F_TC = _BATCH - _F_SC
_GROWS = 120
_GCOLS = 66
_WIN = 7
_FB = 8


def _schedule_segments():
    pairs = np.array(list(combinations(range(_N_ATOMS - 1), 2)), dtype=np.int64)
    pairs = pairs[pairs[:, 1] - pairs[:, 0] != 1]
    buckets = defaultdict(list)
    for idx, (_, j) in enumerate(pairs):
        buckets[j].append(idx)
    chunks = []
    while buckets:
        js = sorted(buckets, key=lambda j: -len(buckets[j]))[:16]
        chunks.append([buckets[j].pop() for j in js])
        for j in js:
            if not buckets[j]:
                del buckets[j]
    si, sj = [], []
    for chunk in chunks:
        ci = [int(pairs[s, 0]) for s in chunk]
        cj = [int(pairs[s, 1]) for s in chunk]
        for lane in range(len(chunk), 16):
            ci.append(0)
            cj.append(100 + lane)
        si.extend(ci)
        sj.extend(cj)
    return (np.asarray(si, np.int32), np.asarray(sj, np.int32), len(chunks))


_SEG_I, _SEG_J, _NCHUNK = _schedule_segments()


def _rsqrt16(v):
    y = plsc.bitcast(jnp.int32(0x5F3759DF) - (plsc.bitcast(v, jnp.int32) >> 1),
                     jnp.float32)
    for _ in range(2):
        y = y * (1.5 - 0.5 * v * y * y)
    return y


def _asin16(x):
    ax = jnp.abs(x)
    p = jnp.full((16,), -0.0012624911, jnp.float32)
    for c in (0.0066700901, -0.0170881256, 0.0308918810,
              -0.0501743046, 0.0889789874, -0.2145988016, 1.5707963050):
        p = p * ax + jnp.float32(c)
    t = jnp.maximum(1.0 - ax, 0.0)
    s = t * _rsqrt16(t)
    return jnp.sign(x) * (jnp.float32(1.5707963267948966) - s * p)


def _cross(a, b):
    return (a[1] * b[2] - a[2] * b[1],
            a[2] * b[0] - a[0] * b[2],
            a[0] * b[1] - a[1] * b[0])


def _dot(a, b):
    return a[0] * b[0] + a[1] * b[1] + a[2] * b[2]


def _normed(v):
    r = _rsqrt16(_dot(v, v))
    return (v[0] * r, v[1] * r, v[2] * r)


def _sc_stage(segi, segj, xx, xy, xz, zg):
    mesh = plsc.VectorSubcoreMesh(core_axis_name="c", subcore_axis_name="s")

    @functools.partial(
        pl.kernel, mesh=mesh,
        compiler_params=pltpu.CompilerParams(needs_layout_passes=False),
        out_type=jax.ShapeDtypeStruct((_F_SC, _GROWS, _GCOLS), jnp.float32),
        scratch_types=[
            pltpu.VMEM((_NCHUNK * 16,), jnp.int32),
            pltpu.VMEM((_NCHUNK * 16,), jnp.int32),
            pltpu.VMEM((128,), jnp.float32),
            pltpu.VMEM((128,), jnp.float32),
            pltpu.VMEM((128,), jnp.float32),
            pltpu.VMEM((_GROWS, _GCOLS), jnp.float32),
        ],
    )
    def sc_g(segi_hbm, segj_hbm, xx_hbm, xy_hbm, xz_hbm, zg_hbm, g_hbm,
             segi_v, segj_v, xxv, xyv, xzv, gv):
        wid = lax.axis_index("s") * 2 + lax.axis_index("c")
        pltpu.sync_copy(segi_hbm, segi_v)
        pltpu.sync_copy(segj_hbm, segj_v)
        nf = _F_SC // _NW

        def frame_body(fi, carry):
            f = wid * nf + fi
            pltpu.sync_copy(xx_hbm.at[f], xxv)
            pltpu.sync_copy(xy_hbm.at[f], xyv)
            pltpu.sync_copy(xz_hbm.at[f], xzv)
            pltpu.sync_copy(zg_hbm, gv)

            def chunk_body(c, carry2):
                base = c * 16
                ii = segi_v[pl.ds(base, 16)]
                jj = segj_v[pl.ds(base, 16)]
                i1 = ii + 1
                j1 = jj + 1
                pi = tuple(plsc.load_gather(r, [ii]) for r in (xxv, xyv, xzv))
                pi1 = tuple(plsc.load_gather(r, [i1]) for r in (xxv, xyv, xzv))
                pj = tuple(plsc.load_gather(r, [jj]) for r in (xxv, xyv, xzv))
                pj1 = tuple(plsc.load_gather(r, [j1]) for r in (xxv, xyv, xzv))

                dx0 = tuple(pj[a] - pi[a] for a in range(3))
                dx1 = tuple(pj1[a] - pi[a] for a in range(3))
                dx2 = tuple(pj[a] - pi1[a] for a in range(3))
                dx3 = tuple(pj1[a] - pi1[a] for a in range(3))
                r2a = _dot(dx0, dx0)
                r2b = _dot(dx3, dx3)
                w1 = jnp.exp(-r2a)
                w2 = jnp.exp(-r2b)

                ra = _rsqrt16(r2a)
                rb = _rsqrt16(r2b)
                u0 = tuple(dx0[a] * ra for a in range(3))
                u3 = tuple(dx3[a] * rb for a in range(3))
                u1 = _normed(dx1)
                u2 = _normed(dx2)

                c0 = _normed(_cross(u0, u1))
                c1 = _normed(_cross(u1, u3))
                c2 = _normed(_cross(u3, u2))
                c3 = _normed(_cross(u2, u0))
                omega = (_asin16(jnp.clip(_dot(c0, c1), -1.0, 1.0))
                         + _asin16(jnp.clip(_dot(c1, c2), -1.0, 1.0))
                         + _asin16(jnp.clip(_dot(c2, c3), -1.0, 1.0))
                         + _asin16(jnp.clip(_dot(c3, c0), -1.0, 1.0)))
                tj = tuple(pj1[a] - pj[a] for a in range(3))
                ti = tuple(pi1[a] - pi[a] for a in range(3))
                sgn = jnp.sign(_dot(_cross(tj, ti), u0))
                vb = omega * sgn * jnp.float32(_INV2PI * _ISTEP) + jnp.float32(31.5)
                k0 = lax.convert_element_type(vb + 0.5, jnp.int32)

                col64 = jnp.full((16,), 64, jnp.int32)
                plsc.addupdate_scatter(gv, [jj, col64], w1)
                plsc.addupdate_scatter(gv, [j1, col64], w2)
                klo = k0 - _WIN // 2
                d0 = vb - lax.convert_element_type(klo, jnp.float32)
                e = jnp.exp(-(d0 * d0))
                ratio = jnp.exp(2.0 * d0 - 1.0)
                decay = jnp.full((16,), math.exp(-2.0), jnp.float32)
                for kk in range(_WIN):
                    k = klo + kk
                    valid = (k >= 0) & (k <= 63)
                    plsc.addupdate_scatter(gv, [jj, k], w1 * e, mask=valid)
                    plsc.addupdate_scatter(gv, [j1, k], w2 * e, mask=valid)
                    if kk < _WIN - 1:
                        e = e * ratio
                        ratio = ratio * decay
                return carry2

            lax.fori_loop(0, _NCHUNK, chunk_body, 0)
            pltpu.sync_copy(gv, g_hbm.at[f])
            return carry

        lax.fori_loop(0, nf, frame_body, 0)

    return sc_g(segi, segj, xx, xy, xz, zg)


def _shl_lane(m):
    return jnp.concatenate([m[:, 1:], m[:, :1]], axis=1)


def _shl_sub(m):
    return jnp.concatenate([m[1:, :], m[:1, :]], axis=0)


def _shr_lane_row(v):
    return jnp.concatenate([jnp.zeros((1, 1), jnp.float32), v[:, :-1]], axis=1)


def _asin_tc(x):
    ax = jnp.abs(x)
    p = jnp.float32(-0.0012624911)
    for c in (0.0066700901, -0.0170881256, 0.0308918810,
              -0.0501743046, 0.0889789874, -0.2145988016, 1.5707963050):
        p = p * ax + jnp.float32(c)
    r = jnp.float32(1.5707963267948966) - jnp.sqrt(jnp.maximum(1.0 - ax, 0.0)) * p
    return jnp.sign(x) * r


def _cross_tc(a, b):
    return (a[1] * b[2] - a[2] * b[1],
            a[2] * b[0] - a[0] * b[2],
            a[0] * b[1] - a[1] * b[0])


def _norm3_tc(v):
    r = lax.rsqrt(v[0] * v[0] + v[1] * v[1] + v[2] * v[2])
    return (v[0] * r, v[1] * r, v[2] * r)


def _dot3_tc(a, b):
    return a[0] * b[0] + a[1] * b[1] + a[2] * b[2]


_NR = 104


def _dense_body(xt_ref, nodef_ref, basis_ref, out_ref, gt_ref):
    for fr in range(2):
        msg = _dense_frame(xt_ref[fr], basis_ref, gt_ref)
        sl = slice(fr * _N_ATOMS, (fr + 1) * _N_ATOMS)
        out_ref[sl, :] = nodef_ref[sl, :] + msg[:_N_ATOMS, :]


def _dense_frame(xr, basis_ref, gt_ref):
    cols = [jnp.transpose(jnp.broadcast_to(xr[c:c + 1, :], (128, 128)))[:_NR, :]
            for c in range(3)]
    rows = [jnp.broadcast_to(xr[c:c + 1, :], (_NR, 128)) for c in range(3)]
    d = [rows[c] - cols[c] for c in range(3)]
    r2 = d[0] * d[0] + d[1] * d[1] + d[2] * d[2]
    w = jnp.exp(-r2)
    inv = lax.rsqrt(r2)
    ua = tuple(d[c] * inv for c in range(3))
    ub = tuple(_shl_lane(u) for u in ua)
    uc = tuple(_shl_sub(u) for u in ua)
    ud = tuple(_shl_sub(u) for u in ub)

    c0 = _norm3_tc(_cross_tc(ua, ub))
    c1 = _norm3_tc(_cross_tc(ub, ud))
    c2 = _norm3_tc(_cross_tc(ud, uc))
    c3 = _norm3_tc(_cross_tc(uc, ua))
    omega = (_asin_tc(jnp.clip(_dot3_tc(c0, c1), -1.0, 1.0))
             + _asin_tc(jnp.clip(_dot3_tc(c1, c2), -1.0, 1.0))
             + _asin_tc(jnp.clip(_dot3_tc(c2, c3), -1.0, 1.0))
             + _asin_tc(jnp.clip(_dot3_tc(c3, c0), -1.0, 1.0)))

    tj = tuple(_shl_lane(rows[c]) - rows[c] for c in range(3))
    ti = tuple(_shl_sub(cols[c]) - cols[c] for c in range(3))
    sgn = jnp.sign(_dot3_tc(_cross_tc(tj, ti), ua))
    wr = omega * sgn * jnp.float32(_INV2PI)

    ii = lax.broadcasted_iota(jnp.int32, (_NR, 128), 0)
    jj = lax.broadcasted_iota(jnp.int32, (_NR, 128), 1)
    mask = (ii + 2 <= jj) & (jj <= 98)
    wrs = jnp.where(mask, wr * jnp.float32(_ISTEP), 0.0)
    w1 = jnp.where(mask, w, 0.0)
    w2 = jnp.where(mask, _shl_sub(_shl_lane(w)), 0.0)

    s1 = jnp.sum(w1, axis=0, keepdims=True)
    s2 = jnp.sum(w2, axis=0, keepdims=True)
    denom = s1 + _shr_lane_row(s2)
    dinv = jnp.where(denom > 0, jnp.float32(1.0 / 1.12) / denom, 0.0)

    onesr = jnp.ones((1, _NR), jnp.float32)
    cdims = (((1,), (0,)), ((), ()))
    for k in range(_BINS):
        dk = wrs - jnp.float32(_CKS[k])
        e = jnp.exp(-(dk * dk))
        r1 = lax.dot_general(onesr, w1 * e, cdims,
                             preferred_element_type=jnp.float32)
        r2v = lax.dot_general(onesr, w2 * e, cdims,
                              preferred_element_type=jnp.float32)
        gt_ref[k:k + 1, :] = r1 + _shr_lane_row(r2v)

    gt = gt_ref[...] * dinv
    return lax.dot_general(gt, basis_ref[...], (((0,), (0,)), ((), ())),
                           preferred_element_type=jnp.float32)


def _tc_finish_body(prev_ref, g_ref, nodef_ref, basis_ref, out_ref):
    del prev_ref
    for i in range(_FB):
        g = g_ref[i]
        den = g[:, 64:65]
        dinv = jnp.where(den > 0, jnp.float32(1.0 / 1.12) / den, 0.0)
        gs = g[:, :_BINS] * dinv
        msg = lax.dot_general(gs, basis_ref[...], (((1,), (0,)), ((), ())),
                              preferred_element_type=jnp.float32)
        sl = slice(i * _N_ATOMS, (i + 1) * _N_ATOMS)
        out_ref[sl, :] = nodef_ref[sl, :] + msg[:_N_ATOMS, :]


@jax.jit
def kernel(x, invariant_node_features, basis):
    xr3 = x.reshape(_BATCH, _N_ATOMS, 3)
    basis2 = basis[0, 0]

    xpad = jnp.pad(xr3[:_F_SC], ((0, 0), (0, 28), (0, 0)))
    xx, xy, xz = xpad[:, :, 0], xpad[:, :, 1], xpad[:, :, 2]
    zg = jnp.zeros((_GROWS, _GCOLS), jnp.float32)
    g = _sc_stage(jnp.asarray(_SEG_I), jnp.asarray(_SEG_J), xx, xy, xz, zg)

    xt = jnp.transpose(xr3[_F_SC:], (0, 2, 1))
    xt = jnp.pad(xt, ((0, 0), (0, 0), (0, 128 - _N_ATOMS)))
    msg_tc = pl.pallas_call(
        _dense_body,
        grid=(_F_TC // 2,),
        in_specs=[
            pl.BlockSpec((2, 3, 128), lambda b: (b, 0, 0)),
            pl.BlockSpec((2 * _N_ATOMS, _N_FEATURES),
                         lambda b: (b + _F_SC // 2, 0)),
            pl.BlockSpec((_BINS, _N_FEATURES), lambda b: (0, 0)),
        ],
        out_specs=pl.BlockSpec((2 * _N_ATOMS, _N_FEATURES),
                               lambda b: (b + _F_SC // 2, 0)),
        out_shape=jax.ShapeDtypeStruct((_BATCH * _N_ATOMS, _N_FEATURES),
                                       jnp.float32),
        scratch_shapes=[pltpu.VMEM((_BINS, 128), jnp.float32)],
    )(xt, invariant_node_features, basis2)

    out = pl.pallas_call(
        _tc_finish_body,
        grid=(_F_SC // _FB,),
        in_specs=[
            pl.BlockSpec(memory_space=pl.ANY),
            pl.BlockSpec((_FB, _GROWS, _GCOLS), lambda b: (b, 0, 0)),
            pl.BlockSpec((_FB * _N_ATOMS, _N_FEATURES), lambda b: (b, 0)),
            pl.BlockSpec((_BINS, _N_FEATURES), lambda b: (0, 0)),
        ],
        out_specs=pl.BlockSpec((_FB * _N_ATOMS, _N_FEATURES), lambda b: (b, 0)),
        out_shape=jax.ShapeDtypeStruct((_BATCH * _N_ATOMS, _N_FEATURES),
                                       jnp.float32),
        input_output_aliases={0: 0},
    )(msg_tc, g, invariant_node_features, basis2)

    return out
